# baseline (speedup 1.0000x reference)
import functools

import jax
import jax.numpy as jnp
from jax import lax
from jax.experimental import pallas as pl
from jax.experimental.pallas import tpu as pltpu
from jax.experimental.pallas import tpu_sc as plsc

N = 100000
E = 3200000
F = 16
G = 256
EPS = 1e-5

NC = 2
NS = 16
NW = NC * NS
EW = E // NW
C = 2000
NCH = EW // C
SUB = 125
NSUB = C // SUB
NSL = N // NS

E8 = E // 8
BLKE = 2000
NBE = E8 // BLKE
BLKN = 4000
NBN = N // BLKN


def _bd(w):
    return jnp.kron(jnp.eye(8, dtype=w.dtype), w)


def _k0_body(x_ref, b_ref, u_ref, wa_ref, wb_ref, wd_ref, a_ref, bt_ref):
    x = x_ref[...]
    oh = (lax.broadcasted_iota(jnp.int32, (BLKN, G), 1) == b_ref[...]).astype(jnp.float32)
    uw = jnp.dot(u_ref[...], wd_ref[...], preferred_element_type=jnp.float32)
    a_ref[...] = (jnp.dot(x, wa_ref[...], preferred_element_type=jnp.float32)
                  + jnp.dot(oh, uw, preferred_element_type=jnp.float32))
    bt_ref[...] = jnp.dot(x, wb_ref[...], preferred_element_type=jnp.float32)


def _node_tables(x, batch2, u, waT, wbT, wdT):
    return pl.pallas_call(
        _k0_body,
        grid=(NBN,),
        in_specs=[
            pl.BlockSpec((BLKN, F), lambda i: (i, 0)),
            pl.BlockSpec((BLKN, 1), lambda i: (i, 0)),
            pl.BlockSpec((G, F), lambda i: (0, 0)),
            pl.BlockSpec((F, F), lambda i: (0, 0)),
            pl.BlockSpec((F, F), lambda i: (0, 0)),
            pl.BlockSpec((F, F), lambda i: (0, 0)),
        ],
        out_specs=[pl.BlockSpec((BLKN, F), lambda i: (i, 0)),
                   pl.BlockSpec((BLKN, F), lambda i: (i, 0))],
        out_shape=[jax.ShapeDtypeStruct((N, F), jnp.float32),
                   jax.ShapeDtypeStruct((N, F), jnp.float32)],
    )(x, batch2, u, waT, wbT, wdT)


def _k1_body(a_hbm, b_hbm, row_hbm, col_hbm, hp_hbm, c_hbm,
             idxr, idxc, ga, gb, htT, ones, zflat, sema, semb, semc, c_sh):
    scid = lax.axis_index("c")
    sid = lax.axis_index("s")
    wid = sid * NC + scid
    base = wid * EW
    iota16 = jax.lax.broadcasted_iota(jnp.int32, (F,), 0)

    @plsc.parallel_loop(0, 128 // F, unroll=8)
    def _ofill(i):
        ones[pl.ds(i * F, F)] = jnp.full((F,), 1.0, jnp.float32)

    @plsc.parallel_loop(0, 1024 // F, unroll=8)
    def _zcfill(i):
        zflat[pl.ds(i * F, F)] = jnp.zeros((F,), jnp.float32)

    cstart = (sid * NSL) // 8 * 8
    for z in range(6):
        pltpu.sync_copy(zflat.at[pl.ds(0, 1000)],
                        c_sh.at[pl.ds(cstart + z * 1000, 1000)])
    pltpu.sync_copy(zflat.at[pl.ds(0, 256)], c_sh.at[pl.ds(cstart + 6000, 256)])
    plsc.subcore_barrier()

    @pl.loop(0, NCH)
    def _chunk(ci):
        off = base + ci * C
        osub = pl.multiple_of(off // SUB, 8)
        pltpu.sync_copy(row_hbm.at[pl.ds(osub, NSUB)], idxr)
        pltpu.sync_copy(col_hbm.at[pl.ds(osub, NSUB)], idxc)
        das = []
        for j in range(NSUB):
            das.append(pltpu.async_copy(
                a_hbm.at[idxr.at[j]], ga.at[pl.ds(j * SUB, SUB)], sema))
            das.append(pltpu.async_copy(
                b_hbm.at[idxc.at[j]], gb.at[pl.ds(j * SUB, SUB)], semb))
            das.append(pltpu.async_copy(
                ones.at[pl.ds(0, SUB)], c_sh.at[idxc.at[j]], semc, add=True))
        for d in das:
            d.wait()

        @plsc.parallel_loop(0, C, unroll=8)
        def _row(i):
            val = ga[i] + gb[i]
            plsc.store_scatter(htT, [iota16, jnp.full((F,), i, jnp.int32)],
                               val)

        pltpu.sync_copy(htT, hp_hbm.at[:, pl.ds(off, C)])

    plsc.subcore_barrier()
    pltpu.sync_copy(c_sh.at[pl.ds(cstart, 6256)],
                    c_hbm.at[scid, pl.ds(cstart, 6256)])


def _gather_hp(A, B, row3, col3):
    mesh = plsc.VectorSubcoreMesh(core_axis_name="c", subcore_axis_name="s")
    return pl.kernel(
        _k1_body,
        out_type=(jax.ShapeDtypeStruct((F, E), jnp.float32),
                  jax.ShapeDtypeStruct((NC, N), jnp.float32)),
        mesh=mesh,
        compiler_params=pltpu.CompilerParams(use_tc_tiling_on_sc=False, needs_layout_passes=False),
        scratch_types=[
            pltpu.VMEM((NSUB, SUB), jnp.int32),
            pltpu.VMEM((NSUB, SUB), jnp.int32),
            pltpu.VMEM((C, F), jnp.float32),
            pltpu.VMEM((C, F), jnp.float32),
            pltpu.VMEM((F, C), jnp.float32),
            pltpu.VMEM((128,), jnp.float32),
            pltpu.VMEM((1024,), jnp.float32),
            pltpu.SemaphoreType.DMA,
            pltpu.SemaphoreType.DMA,
            pltpu.SemaphoreType.DMA,
            pltpu.VMEM_SHARED((N,), jnp.float32),
        ],
    )(A, B, row3, col3)


BLKC = 6400
NBC = E // BLKC


def _k2a_body(hp_ref, ea_ref, wc_ref, h_ref, sum_ref, sq_ref):
    i = pl.program_id(0)
    h = hp_ref[...] + jnp.dot(wc_ref[...], ea_ref[...],
                              preferred_element_type=jnp.float32)
    h_ref[...] = h

    @pl.when(i == 0)
    def _():
        sum_ref[...] = jnp.zeros_like(sum_ref)
        sq_ref[...] = jnp.zeros_like(sq_ref)

    sum_ref[...] += h
    sq_ref[...] += h * h


def _edge_h(hpT, eaT, wc):
    return pl.pallas_call(
        _k2a_body,
        grid=(NBC,),
        in_specs=[
            pl.BlockSpec((F, BLKC), lambda i: (0, i)),
            pl.BlockSpec((F, BLKC), lambda i: (0, i)),
            pl.BlockSpec((F, F), lambda i: (0, 0)),
        ],
        out_specs=[pl.BlockSpec((F, BLKC), lambda i: (0, i)),
                   pl.BlockSpec((F, BLKC), lambda i: (0, 0)),
                   pl.BlockSpec((F, BLKC), lambda i: (0, 0))],
        out_shape=[jax.ShapeDtypeStruct((F, E), jnp.float32),
                   jax.ShapeDtypeStruct((F, BLKC), jnp.float32),
                   jax.ShapeDtypeStruct((F, BLKC), jnp.float32)],
    )(hpT, eaT, wc)


def _k2b_body(h_ref, s_ref, t_ref, w2_ref, b2_ref, eo_ref):
    r = jnp.maximum(h_ref[...] * s_ref[...] + t_ref[...], 0.0)
    eo_ref[...] = jnp.dot(w2_ref[...], r,
                          preferred_element_type=jnp.float32) + b2_ref[...]


def _edge_out(hT, scol, tcol, w2, b2col):
    return pl.pallas_call(
        _k2b_body,
        grid=(NBC,),
        in_specs=[
            pl.BlockSpec((F, BLKC), lambda i: (0, i)),
            pl.BlockSpec((F, 1), lambda i: (0, 0)),
            pl.BlockSpec((F, 1), lambda i: (0, 0)),
            pl.BlockSpec((F, F), lambda i: (0, 0)),
            pl.BlockSpec((F, 1), lambda i: (0, 0)),
        ],
        out_specs=pl.BlockSpec((F, BLKC), lambda i: (0, i)),
        out_shape=jax.ShapeDtypeStruct((F, E), jnp.float32),
    )(hT, scol, tcol, w2, b2col)


C3 = 1000
H3 = 500
NCH3 = EW // C3
NSUB3 = H3 // SUB


def _k3_body(h_hbm, col_hbm, s_hbm, t_hbm, sout_hbm,
             idxc, hv, hvT, sv_v, tv_v, s_sh):
    scid = lax.axis_index("c")
    sid = lax.axis_index("s")
    base = (scid * NS + sid) * EW
    iota16 = jax.lax.broadcasted_iota(jnp.int32, (F,), 0)

    pltpu.sync_copy(s_hbm, sv_v)
    pltpu.sync_copy(t_hbm, tv_v)
    sv = sv_v[...]
    tv = tv_v[...]

    @plsc.parallel_loop(0, H3, unroll=8)
    def _zfill(i):
        hv[i] = jnp.zeros((F,), jnp.float32)

    for z in range(13):
        zo = min(z * H3, NSL - H3)
        pltpu.sync_copy(hv, s_sh.at[pl.ds(sid * NSL + zo, H3)])
    plsc.subcore_barrier()

    @pl.loop(0, NCH3)
    def _chunk(ci):
        off = base + ci * C3
        pltpu.sync_copy(h_hbm.at[:, pl.ds(off, C3)], hvT)
        pltpu.sync_copy(col_hbm.at[pl.ds(pl.multiple_of(off // SUB, 8), C3 // SUB)],
                        idxc)
        for h2 in range(2):

            @plsc.parallel_loop(0, H3, unroll=8)
            def _row(i):
                hcol = plsc.load_gather(
                    hvT, [iota16, jnp.full((F,), h2 * H3 + i, jnp.int32)])
                hv[i] = jnp.maximum(hcol * sv + tv, 0.0)

            for j in range(NSUB3):
                pltpu.sync_copy(hv.at[pl.ds(j * SUB, SUB)],
                                s_sh.at[idxc.at[h2 * NSUB3 + j]], add=True)

    plsc.subcore_barrier()
    pltpu.sync_copy(s_sh.at[pl.ds(sid * NSL, NSL)],
                    sout_hbm.at[scid, pl.ds(sid * NSL, NSL)])


def _scatter_r(h, col3, s16, t16):
    mesh = plsc.VectorSubcoreMesh(core_axis_name="c", subcore_axis_name="s")
    return pl.kernel(
        _k3_body,
        out_type=jax.ShapeDtypeStruct((NC, N, F), jnp.float32),
        mesh=mesh,
        compiler_params=pltpu.CompilerParams(use_tc_tiling_on_sc=False, needs_layout_passes=False),
        scratch_types=[
            pltpu.VMEM((C3 // SUB, SUB), jnp.int32),
            pltpu.VMEM((H3, F), jnp.float32),
            pltpu.VMEM((F, C3), jnp.float32),
            pltpu.VMEM((F,), jnp.float32),
            pltpu.VMEM((F,), jnp.float32),
            pltpu.VMEM_SHARED((N, F), jnp.float32),
        ],
    )(h, col3, s16, t16)


def _k4_body(x_ref, s0_ref, s1_ref, c0_ref, c1_ref, b2_ref, bt_ref, u_ref,
             mats_ref, vecs_ref,
             xo_ref, uo_ref,
             nsum, nsq, xa, sg, ecg, ncnt):
    i = pl.program_id(0)
    nwaT = mats_ref[0:16, :]
    nwbT = mats_ref[16:32, :]
    nw2T = mats_ref[32:48, :]
    ew2T = mats_ref[48:64, :]
    gwaT = mats_ref[64:80, :]
    gwbT = mats_ref[80:96, :]
    gwcT = mats_ref[96:112, :]
    n_b1 = vecs_ref[0:1, :]
    n_b2 = vecs_ref[1:2, :]
    n_g1 = vecs_ref[2:3, :]
    n_be1 = vecs_ref[3:4, :]
    e_b2 = vecs_ref[4:5, :]
    g_b1 = vecs_ref[5:6, :]
    g_g1 = vecs_ref[6:7, :]
    g_be1 = vecs_ref[7:8, :]

    bT = bt_ref[0]
    ohT = (lax.broadcasted_iota(jnp.int32, (G, BLKN), 0) == bT).astype(jnp.float32)

    def node_hidden():
        s_blk = s0_ref[...] + s1_ref[...]
        c_blk = jnp.broadcast_to(c0_ref[...] + c1_ref[...], (BLKN, F))
        e_aggr = ((jnp.dot(s_blk, ew2T, preferred_element_type=jnp.float32)
                   + c_blk * e_b2) / jnp.maximum(c_blk, 1.0))
        oh = (lax.broadcasted_iota(jnp.int32, (BLKN, G), 1)
              == b2_ref[...]).astype(jnp.float32)
        hn = (jnp.dot(x_ref[...], nwaT, preferred_element_type=jnp.float32)
              + jnp.dot(e_aggr, nwbT, preferred_element_type=jnp.float32)
              + jnp.dot(oh, jnp.dot(u_ref[...], mats_ref[112:128, :],
                                    preferred_element_type=jnp.float32),
                        preferred_element_type=jnp.float32)
              + n_b1)
        return hn, s_blk, c_blk

    @pl.when(i == 0)
    def _():
        nsum[...] = jnp.zeros_like(nsum)
        nsq[...] = jnp.zeros_like(nsq)
        xa[...] = jnp.zeros_like(xa)
        sg[...] = jnp.zeros_like(sg)
        ecg[...] = jnp.zeros_like(ecg)
        ncnt[...] = jnp.zeros_like(ncnt)

    @pl.when(i < NBN)
    def _phase_a():
        hn, s_blk, c_blk = node_hidden()
        nsum[...] += jnp.sum(hn, axis=0, keepdims=True)
        nsq[...] += jnp.sum(hn * hn, axis=0, keepdims=True)
        sg[...] += jnp.dot(ohT, s_blk, preferred_element_type=jnp.float32)
        ecg[...] += jnp.dot(ohT, c_blk, preferred_element_type=jnp.float32)
        ncnt[...] += jnp.dot(ohT, jnp.ones((BLKN, F), jnp.float32),
                             preferred_element_type=jnp.float32)

    @pl.when(i >= NBN)
    def _phase_b():
        hn, _, _ = node_hidden()
        mn = nsum[...] / N
        vr = nsq[...] / N - mn * mn
        sn = n_g1 * lax.rsqrt(vr + EPS)
        tn = n_be1 - mn * sn
        xo = jnp.dot(jnp.maximum(hn * sn + tn, 0.0), nw2T,
                     preferred_element_type=jnp.float32) + n_b2
        xo_ref[...] = xo
        w = jnp.where(i < 2 * NBN, 1.0, 0.0).astype(jnp.float32)
        xa[...] += w * jnp.dot(ohT, xo, preferred_element_type=jnp.float32)

    @pl.when(i == 2 * NBN)
    def _phase_g():
        x_aggr = xa[...] / jnp.maximum(ncnt[...], 1.0)
        e_aggr_g = ((jnp.dot(sg[...], ew2T, preferred_element_type=jnp.float32)
                     + ecg[...] * e_b2) / jnp.maximum(ecg[...], 1.0))
        go = (jnp.dot(u_ref[...], gwaT, preferred_element_type=jnp.float32)
              + jnp.dot(x_aggr, gwbT, preferred_element_type=jnp.float32)
              + jnp.dot(e_aggr_g, gwcT, preferred_element_type=jnp.float32)
              + g_b1)
        mg = jnp.sum(go, axis=0, keepdims=True) / G
        vg = jnp.sum(go * go, axis=0, keepdims=True) / G - mg * mg
        uo_ref[...] = jnp.maximum(
            (go - mg) * lax.rsqrt(vg + EPS) * g_g1 + g_be1, 0.0)


def _node_global(x, S, c2, batch2, batchT, u, mats, vecs):
    S0 = S[0]
    S1 = S[1]
    c0 = c2[0].reshape(N, 1)
    c1 = c2[1].reshape(N, 1)
    nb = lambda i: (i % NBN, 0)
    return pl.pallas_call(
        _k4_body,
        grid=(2 * NBN + 1,),
        in_specs=[
            pl.BlockSpec((BLKN, F), nb),
            pl.BlockSpec((BLKN, F), nb),
            pl.BlockSpec((BLKN, F), nb),
            pl.BlockSpec((BLKN, 1), nb),
            pl.BlockSpec((BLKN, 1), nb),
            pl.BlockSpec((BLKN, 1), nb),
            pl.BlockSpec((1, 1, BLKN), lambda i: (i % NBN, 0, 0)),
            pl.BlockSpec((G, F), lambda i: (0, 0)),
            pl.BlockSpec((128, F), lambda i: (0, 0)),
            pl.BlockSpec((8, F), lambda i: (0, 0)),
        ],
        out_specs=[pl.BlockSpec((BLKN, F), nb),
                   pl.BlockSpec((G, F), lambda i: (0, 0))],
        out_shape=[jax.ShapeDtypeStruct((N, F), jnp.float32),
                   jax.ShapeDtypeStruct((G, F), jnp.float32)],
        scratch_shapes=[
            pltpu.VMEM((1, F), jnp.float32),
            pltpu.VMEM((1, F), jnp.float32),
            pltpu.VMEM((G, F), jnp.float32),
            pltpu.VMEM((G, F), jnp.float32),
            pltpu.VMEM((G, F), jnp.float32),
            pltpu.VMEM((G, F), jnp.float32),
        ],
    )(x, S0, S1, c0, c1, batch2, batchT, u, mats, vecs)


def kernel(x, edge_index, edge_attr, u, batch,
           e_W1, e_b1, e_g1, e_be1, e_W2, e_b2,
           n_W1, n_b1, n_g1, n_be1, n_W2, n_b2,
           g_W1, g_b1, g_g1, g_be1):
    row = edge_index[0].astype(jnp.int32)
    col = edge_index[1].astype(jnp.int32)
    batch = batch.astype(jnp.int32)

    eWaT = e_W1[:, 0:16].T
    eWbT = e_W1[:, 16:32].T
    eWcT = e_W1[:, 32:48].T
    eWdT = e_W1[:, 48:64].T

    batch2 = batch.reshape(N, 1)
    batchT = batch.reshape(NBN, 1, BLKN)
    row3 = row.reshape(E // SUB, SUB)
    col3 = col.reshape(E // SUB, SUB)

    A, B = _node_tables(x, batch2, u, eWaT, eWbT, eWdT)

    hp, c2 = _gather_hp(A, B, row3, col3)

    eaT = edge_attr.T
    hT, hsum, hsq = _edge_h(hp, eaT, e_W1[:, 32:48])

    hsum16 = jnp.sum(hsum, axis=1)
    hsq16 = jnp.sum(hsq, axis=1)
    mean = hsum16 / E
    var = hsq16 / E - mean * mean
    s16 = e_g1 * lax.rsqrt(var + EPS)
    t16 = e_be1 - mean * s16

    eoT = _edge_out(hT, s16.reshape(F, 1), t16.reshape(F, 1), e_W2,
                    e_b2.reshape(F, 1))
    edge_out = eoT.T

    S = _scatter_r(hT, col3, s16, t16)

    mats = jnp.concatenate([
        n_W1[:, 0:16].T, n_W1[:, 16:32].T, n_W2.T, e_W2.T,
        g_W1[:, 0:16].T, g_W1[:, 16:32].T, g_W1[:, 32:48].T,
        n_W1[:, 32:48].T,
    ], axis=0)
    vecs = jnp.stack([n_b1, n_b2, n_g1, n_be1, e_b2, g_b1, g_g1, g_be1])
    x_out, u_out = _node_global(x, S, c2, batch2, batchT, u, mats, vecs)

    return (x_out, edge_out, u_out)


# speedup vs baseline: 2.6682x; 2.6682x over previous
import functools

import jax
import jax.numpy as jnp
from jax import lax
from jax.experimental import pallas as pl
from jax.experimental.pallas import tpu as pltpu
from jax.experimental.pallas import tpu_sc as plsc

N = 100000
E = 3200000
F = 16
G = 256
EPS = 1e-5

NC = 2
NS = 16
NW = NC * NS
EW = E // NW
C = 2000
NCH = EW // C
SUB = 125
NSUB = C // SUB
NSL = N // NS

E8 = E // 8
BLKE = 2000
NBE = E8 // BLKE
BLKN = 4000
NBN = N // BLKN


def _bd(w):
    return jnp.kron(jnp.eye(8, dtype=w.dtype), w)


def _k0_body(x_ref, b_ref, u_ref, wa_ref, wb_ref, wd_ref, a_ref, bt_ref):
    x = x_ref[...]
    oh = (lax.broadcasted_iota(jnp.int32, (BLKN, G), 1) == b_ref[...]).astype(jnp.float32)
    uw = jnp.dot(u_ref[...], wd_ref[...], preferred_element_type=jnp.float32)
    a_ref[...] = (jnp.dot(x, wa_ref[...], preferred_element_type=jnp.float32)
                  + jnp.dot(oh, uw, preferred_element_type=jnp.float32))
    bt_ref[...] = jnp.dot(x, wb_ref[...], preferred_element_type=jnp.float32)


def _node_tables(x, batch2, u, waT, wbT, wdT):
    return pl.pallas_call(
        _k0_body,
        grid=(NBN,),
        in_specs=[
            pl.BlockSpec((BLKN, F), lambda i: (i, 0)),
            pl.BlockSpec((BLKN, 1), lambda i: (i, 0)),
            pl.BlockSpec((G, F), lambda i: (0, 0)),
            pl.BlockSpec((F, F), lambda i: (0, 0)),
            pl.BlockSpec((F, F), lambda i: (0, 0)),
            pl.BlockSpec((F, F), lambda i: (0, 0)),
        ],
        out_specs=[pl.BlockSpec((BLKN, F), lambda i: (i, 0)),
                   pl.BlockSpec((BLKN, F), lambda i: (i, 0))],
        out_shape=[jax.ShapeDtypeStruct((N, F), jnp.float32),
                   jax.ShapeDtypeStruct((N, F), jnp.float32)],
    )(x, batch2, u, waT, wbT, wdT)


def _k1_body(a_hbm, b_hbm, row_hbm, col_hbm, hp_hbm, c_hbm,
             idxr, idxc, ga, gb, htT, ones, zflat, sema, semb, semc, c_sh):
    scid = lax.axis_index("c")
    sid = lax.axis_index("s")
    wid = sid * NC + scid
    base = wid * EW
    iota16 = jax.lax.broadcasted_iota(jnp.int32, (F,), 0)

    @plsc.parallel_loop(0, 128 // F, unroll=8)
    def _ofill(i):
        ones[pl.ds(i * F, F)] = jnp.full((F,), 1.0, jnp.float32)

    @plsc.parallel_loop(0, 1024 // F, unroll=8)
    def _zcfill(i):
        zflat[pl.ds(i * F, F)] = jnp.zeros((F,), jnp.float32)

    cstart = (sid * NSL) // 8 * 8
    for z in range(6):
        pltpu.sync_copy(zflat.at[pl.ds(0, 1000)],
                        c_sh.at[pl.ds(cstart + z * 1000, 1000)])
    pltpu.sync_copy(zflat.at[pl.ds(0, 256)], c_sh.at[pl.ds(cstart + 6000, 256)])
    plsc.subcore_barrier()

    @pl.loop(0, NCH)
    def _chunk(ci):
        off = base + ci * C
        osub = pl.multiple_of(off // SUB, 8)
        pltpu.sync_copy(row_hbm.at[pl.ds(osub, NSUB)], idxr)
        pltpu.sync_copy(col_hbm.at[pl.ds(osub, NSUB)], idxc)
        das = []
        for j in range(NSUB):
            das.append(pltpu.async_copy(
                a_hbm.at[idxr.at[j]], ga.at[pl.ds(j * SUB, SUB)], sema))
            das.append(pltpu.async_copy(
                b_hbm.at[idxc.at[j]], gb.at[pl.ds(j * SUB, SUB)], semb))
            das.append(pltpu.async_copy(
                ones.at[pl.ds(0, SUB)], c_sh.at[idxc.at[j]], semc, add=True))
        for d in das:
            d.wait()

        @plsc.parallel_loop(0, C, unroll=8)
        def _row(i):
            val = ga[i] + gb[i]
            plsc.store_scatter(htT, [iota16, jnp.full((F,), i, jnp.int32)],
                               val)

        pltpu.sync_copy(htT, hp_hbm.at[:, pl.ds(off, C)])

    plsc.subcore_barrier()
    pltpu.sync_copy(c_sh.at[pl.ds(cstart, 6256)],
                    c_hbm.at[scid, pl.ds(cstart, 6256)])


def _gather_hp(A, B, row3, col3):
    mesh = plsc.VectorSubcoreMesh(core_axis_name="c", subcore_axis_name="s")
    return pl.kernel(
        _k1_body,
        out_type=(jax.ShapeDtypeStruct((F, E), jnp.float32),
                  jax.ShapeDtypeStruct((NC, N), jnp.float32)),
        mesh=mesh,
        compiler_params=pltpu.CompilerParams(use_tc_tiling_on_sc=False, needs_layout_passes=False),
        scratch_types=[
            pltpu.VMEM((NSUB, SUB), jnp.int32),
            pltpu.VMEM((NSUB, SUB), jnp.int32),
            pltpu.VMEM((C, F), jnp.float32),
            pltpu.VMEM((C, F), jnp.float32),
            pltpu.VMEM((F, C), jnp.float32),
            pltpu.VMEM((128,), jnp.float32),
            pltpu.VMEM((1024,), jnp.float32),
            pltpu.SemaphoreType.DMA,
            pltpu.SemaphoreType.DMA,
            pltpu.SemaphoreType.DMA,
            pltpu.VMEM_SHARED((N,), jnp.float32),
        ],
    )(A, B, row3, col3)


EC = E // 128
CB = 200
NBC = EC // CB


def _k2a_body(hp_ref, ea_ref, wc_ref, h_ref, sum_ref, sq_ref):
    i = pl.program_id(0)

    @pl.when(i == 0)
    def _():
        sum_ref[...] = jnp.zeros_like(sum_ref)
        sq_ref[...] = jnp.zeros_like(sq_ref)

    wc = wc_ref[...]
    for c in range(CB):
        h = hp_ref[:, c, :] + jnp.dot(wc, ea_ref[:, c, :],
                                      preferred_element_type=jnp.float32)
        h_ref[:, c, :] = h
        sum_ref[...] += h
        sq_ref[...] += h * h


def _edge_h(hp3, ea3, wc):
    return pl.pallas_call(
        _k2a_body,
        grid=(NBC,),
        in_specs=[
            pl.BlockSpec((F, CB, 128), lambda i: (0, i, 0)),
            pl.BlockSpec((F, CB, 128), lambda i: (0, i, 0)),
            pl.BlockSpec((F, F), lambda i: (0, 0)),
        ],
        out_specs=[pl.BlockSpec((F, CB, 128), lambda i: (0, i, 0)),
                   pl.BlockSpec((F, 128), lambda i: (0, 0)),
                   pl.BlockSpec((F, 128), lambda i: (0, 0))],
        out_shape=[jax.ShapeDtypeStruct((F, EC, 128), jnp.float32),
                   jax.ShapeDtypeStruct((F, 128), jnp.float32),
                   jax.ShapeDtypeStruct((F, 128), jnp.float32)],
    )(hp3, ea3, wc)


def _k2b_body(h_ref, s_ref, t_ref, w2_ref, b2_ref, eo_ref):
    w2 = w2_ref[...]
    s = s_ref[...]
    t = t_ref[...]
    b2 = b2_ref[...]
    for c in range(CB):
        r = jnp.maximum(h_ref[:, c, :] * s + t, 0.0)
        eo_ref[:, c, :] = jnp.dot(w2, r,
                                  preferred_element_type=jnp.float32) + b2


def _edge_out(h3, scol, tcol, w2, b2col):
    return pl.pallas_call(
        _k2b_body,
        grid=(NBC,),
        in_specs=[
            pl.BlockSpec((F, CB, 128), lambda i: (0, i, 0)),
            pl.BlockSpec((F, 1), lambda i: (0, 0)),
            pl.BlockSpec((F, 1), lambda i: (0, 0)),
            pl.BlockSpec((F, F), lambda i: (0, 0)),
            pl.BlockSpec((F, 1), lambda i: (0, 0)),
        ],
        out_specs=pl.BlockSpec((F, CB, 128), lambda i: (0, i, 0)),
        out_shape=jax.ShapeDtypeStruct((F, EC, 128), jnp.float32),
    )(h3, scol, tcol, w2, b2col)


C3 = 1000
H3 = 500
NCH3 = EW // C3
NSUB3 = H3 // SUB


def _k3_body(h_hbm, col_hbm, s_hbm, t_hbm, sout_hbm,
             idxc, hv, hvT, sv_v, tv_v, s_sh):
    scid = lax.axis_index("c")
    sid = lax.axis_index("s")
    base = (scid * NS + sid) * EW
    iota16 = jax.lax.broadcasted_iota(jnp.int32, (F,), 0)

    pltpu.sync_copy(s_hbm, sv_v)
    pltpu.sync_copy(t_hbm, tv_v)
    sv = sv_v[...]
    tv = tv_v[...]

    @plsc.parallel_loop(0, H3, unroll=8)
    def _zfill(i):
        hv[i] = jnp.zeros((F,), jnp.float32)

    for z in range(13):
        zo = min(z * H3, NSL - H3)
        pltpu.sync_copy(hv, s_sh.at[pl.ds(sid * NSL + zo, H3)])
    plsc.subcore_barrier()

    @pl.loop(0, NCH3)
    def _chunk(ci):
        off = base + ci * C3
        pltpu.sync_copy(h_hbm.at[:, pl.ds(off, C3)], hvT)
        pltpu.sync_copy(col_hbm.at[pl.ds(pl.multiple_of(off // SUB, 8), C3 // SUB)],
                        idxc)
        for h2 in range(2):

            @plsc.parallel_loop(0, H3, unroll=8)
            def _row(i):
                hcol = plsc.load_gather(
                    hvT, [iota16, jnp.full((F,), h2 * H3 + i, jnp.int32)])
                hv[i] = jnp.maximum(hcol * sv + tv, 0.0)

            for j in range(NSUB3):
                pltpu.sync_copy(hv.at[pl.ds(j * SUB, SUB)],
                                s_sh.at[idxc.at[h2 * NSUB3 + j]], add=True)

    plsc.subcore_barrier()
    pltpu.sync_copy(s_sh.at[pl.ds(sid * NSL, NSL)],
                    sout_hbm.at[scid, pl.ds(sid * NSL, NSL)])


def _scatter_r(h, col3, s16, t16):
    mesh = plsc.VectorSubcoreMesh(core_axis_name="c", subcore_axis_name="s")
    return pl.kernel(
        _k3_body,
        out_type=jax.ShapeDtypeStruct((NC, N, F), jnp.float32),
        mesh=mesh,
        compiler_params=pltpu.CompilerParams(use_tc_tiling_on_sc=False, needs_layout_passes=False),
        scratch_types=[
            pltpu.VMEM((C3 // SUB, SUB), jnp.int32),
            pltpu.VMEM((H3, F), jnp.float32),
            pltpu.VMEM((F, C3), jnp.float32),
            pltpu.VMEM((F,), jnp.float32),
            pltpu.VMEM((F,), jnp.float32),
            pltpu.VMEM_SHARED((N, F), jnp.float32),
        ],
    )(h, col3, s16, t16)


def _k4_body(x_ref, s0_ref, s1_ref, c0_ref, c1_ref, b2_ref, bt_ref, u_ref,
             mats_ref, vecs_ref,
             xo_ref, uo_ref,
             nsum, nsq, xa, sg, ecg, ncnt):
    i = pl.program_id(0)
    nwaT = mats_ref[0:16, :]
    nwbT = mats_ref[16:32, :]
    nw2T = mats_ref[32:48, :]
    ew2T = mats_ref[48:64, :]
    gwaT = mats_ref[64:80, :]
    gwbT = mats_ref[80:96, :]
    gwcT = mats_ref[96:112, :]
    n_b1 = vecs_ref[0:1, :]
    n_b2 = vecs_ref[1:2, :]
    n_g1 = vecs_ref[2:3, :]
    n_be1 = vecs_ref[3:4, :]
    e_b2 = vecs_ref[4:5, :]
    g_b1 = vecs_ref[5:6, :]
    g_g1 = vecs_ref[6:7, :]
    g_be1 = vecs_ref[7:8, :]

    bT = bt_ref[0]
    ohT = (lax.broadcasted_iota(jnp.int32, (G, BLKN), 0) == bT).astype(jnp.float32)

    def node_hidden():
        s_blk = s0_ref[...] + s1_ref[...]
        c_blk = jnp.broadcast_to(c0_ref[...] + c1_ref[...], (BLKN, F))
        e_aggr = ((jnp.dot(s_blk, ew2T, preferred_element_type=jnp.float32)
                   + c_blk * e_b2) / jnp.maximum(c_blk, 1.0))
        oh = (lax.broadcasted_iota(jnp.int32, (BLKN, G), 1)
              == b2_ref[...]).astype(jnp.float32)
        hn = (jnp.dot(x_ref[...], nwaT, preferred_element_type=jnp.float32)
              + jnp.dot(e_aggr, nwbT, preferred_element_type=jnp.float32)
              + jnp.dot(oh, jnp.dot(u_ref[...], mats_ref[112:128, :],
                                    preferred_element_type=jnp.float32),
                        preferred_element_type=jnp.float32)
              + n_b1)
        return hn, s_blk, c_blk

    @pl.when(i == 0)
    def _():
        nsum[...] = jnp.zeros_like(nsum)
        nsq[...] = jnp.zeros_like(nsq)
        xa[...] = jnp.zeros_like(xa)
        sg[...] = jnp.zeros_like(sg)
        ecg[...] = jnp.zeros_like(ecg)
        ncnt[...] = jnp.zeros_like(ncnt)

    @pl.when(i < NBN)
    def _phase_a():
        hn, s_blk, c_blk = node_hidden()
        nsum[...] += jnp.sum(hn, axis=0, keepdims=True)
        nsq[...] += jnp.sum(hn * hn, axis=0, keepdims=True)
        sg[...] += jnp.dot(ohT, s_blk, preferred_element_type=jnp.float32)
        ecg[...] += jnp.dot(ohT, c_blk, preferred_element_type=jnp.float32)
        ncnt[...] += jnp.dot(ohT, jnp.ones((BLKN, F), jnp.float32),
                             preferred_element_type=jnp.float32)

    @pl.when(i >= NBN)
    def _phase_b():
        hn, _, _ = node_hidden()
        mn = nsum[...] / N
        vr = nsq[...] / N - mn * mn
        sn = n_g1 * lax.rsqrt(vr + EPS)
        tn = n_be1 - mn * sn
        xo = jnp.dot(jnp.maximum(hn * sn + tn, 0.0), nw2T,
                     preferred_element_type=jnp.float32) + n_b2
        xo_ref[...] = xo
        w = jnp.where(i < 2 * NBN, 1.0, 0.0).astype(jnp.float32)
        xa[...] += w * jnp.dot(ohT, xo, preferred_element_type=jnp.float32)

    @pl.when(i == 2 * NBN)
    def _phase_g():
        x_aggr = xa[...] / jnp.maximum(ncnt[...], 1.0)
        e_aggr_g = ((jnp.dot(sg[...], ew2T, preferred_element_type=jnp.float32)
                     + ecg[...] * e_b2) / jnp.maximum(ecg[...], 1.0))
        go = (jnp.dot(u_ref[...], gwaT, preferred_element_type=jnp.float32)
              + jnp.dot(x_aggr, gwbT, preferred_element_type=jnp.float32)
              + jnp.dot(e_aggr_g, gwcT, preferred_element_type=jnp.float32)
              + g_b1)
        mg = jnp.sum(go, axis=0, keepdims=True) / G
        vg = jnp.sum(go * go, axis=0, keepdims=True) / G - mg * mg
        uo_ref[...] = jnp.maximum(
            (go - mg) * lax.rsqrt(vg + EPS) * g_g1 + g_be1, 0.0)


def _node_global(x, S, c2, batch2, batchT, u, mats, vecs):
    S0 = S[0]
    S1 = S[1]
    c0 = c2[0].reshape(N, 1)
    c1 = c2[1].reshape(N, 1)
    nb = lambda i: (i % NBN, 0)
    return pl.pallas_call(
        _k4_body,
        grid=(2 * NBN + 1,),
        in_specs=[
            pl.BlockSpec((BLKN, F), nb),
            pl.BlockSpec((BLKN, F), nb),
            pl.BlockSpec((BLKN, F), nb),
            pl.BlockSpec((BLKN, 1), nb),
            pl.BlockSpec((BLKN, 1), nb),
            pl.BlockSpec((BLKN, 1), nb),
            pl.BlockSpec((1, 1, BLKN), lambda i: (i % NBN, 0, 0)),
            pl.BlockSpec((G, F), lambda i: (0, 0)),
            pl.BlockSpec((128, F), lambda i: (0, 0)),
            pl.BlockSpec((8, F), lambda i: (0, 0)),
        ],
        out_specs=[pl.BlockSpec((BLKN, F), nb),
                   pl.BlockSpec((G, F), lambda i: (0, 0))],
        out_shape=[jax.ShapeDtypeStruct((N, F), jnp.float32),
                   jax.ShapeDtypeStruct((G, F), jnp.float32)],
        scratch_shapes=[
            pltpu.VMEM((1, F), jnp.float32),
            pltpu.VMEM((1, F), jnp.float32),
            pltpu.VMEM((G, F), jnp.float32),
            pltpu.VMEM((G, F), jnp.float32),
            pltpu.VMEM((G, F), jnp.float32),
            pltpu.VMEM((G, F), jnp.float32),
        ],
    )(x, S0, S1, c0, c1, batch2, batchT, u, mats, vecs)


def kernel(x, edge_index, edge_attr, u, batch,
           e_W1, e_b1, e_g1, e_be1, e_W2, e_b2,
           n_W1, n_b1, n_g1, n_be1, n_W2, n_b2,
           g_W1, g_b1, g_g1, g_be1):
    row = edge_index[0].astype(jnp.int32)
    col = edge_index[1].astype(jnp.int32)
    batch = batch.astype(jnp.int32)

    eWaT = e_W1[:, 0:16].T
    eWbT = e_W1[:, 16:32].T
    eWcT = e_W1[:, 32:48].T
    eWdT = e_W1[:, 48:64].T

    batch2 = batch.reshape(N, 1)
    batchT = batch.reshape(NBN, 1, BLKN)
    row3 = row.reshape(E // SUB, SUB)
    col3 = col.reshape(E // SUB, SUB)

    A, B = _node_tables(x, batch2, u, eWaT, eWbT, eWdT)

    hp, c2 = _gather_hp(A, B, row3, col3)

    ea3 = edge_attr.T.reshape(F, EC, 128)
    hp3 = hp.reshape(F, EC, 128)
    h3, hsum, hsq = _edge_h(hp3, ea3, e_W1[:, 32:48])

    hsum16 = jnp.sum(hsum, axis=1)
    hsq16 = jnp.sum(hsq, axis=1)
    mean = hsum16 / E
    var = hsq16 / E - mean * mean
    s16 = e_g1 * lax.rsqrt(var + EPS)
    t16 = e_be1 - mean * s16

    eo3 = _edge_out(h3, s16.reshape(F, 1), t16.reshape(F, 1), e_W2,
                    e_b2.reshape(F, 1))
    edge_out = eo3.reshape(F, E).T

    S = _scatter_r(h3.reshape(F, E), col3, s16, t16)

    mats = jnp.concatenate([
        n_W1[:, 0:16].T, n_W1[:, 16:32].T, n_W2.T, e_W2.T,
        g_W1[:, 0:16].T, g_W1[:, 16:32].T, g_W1[:, 32:48].T,
        n_W1[:, 32:48].T,
    ], axis=0)
    vecs = jnp.stack([n_b1, n_b2, n_g1, n_be1, e_b2, g_b1, g_g1, g_be1])
    x_out, u_out = _node_global(x, S, c2, batch2, batchT, u, mats, vecs)

    return (x_out, edge_out, u_out)


# speedup vs baseline: 2.8284x; 1.0601x over previous
import functools

import jax
import jax.numpy as jnp
from jax import lax
from jax.experimental import pallas as pl
from jax.experimental.pallas import tpu as pltpu
from jax.experimental.pallas import tpu_sc as plsc

N = 100000
E = 3200000
F = 16
G = 256
EPS = 1e-5

NC = 2
NS = 16
NW = NC * NS
EW = E // NW
C = 2000
NCH = EW // C
SUB = 125
NSUB = C // SUB
NSL = N // NS

E8 = E // 8
BLKE = 2000
NBE = E8 // BLKE
BLKN = 4000
NBN = N // BLKN


def _bd(w):
    return jnp.kron(jnp.eye(8, dtype=w.dtype), w)


def _k0_body(x_ref, b_ref, u_ref, wa_ref, wb_ref, wd_ref, a_ref, bt_ref):
    x = x_ref[...]
    oh = (lax.broadcasted_iota(jnp.int32, (BLKN, G), 1) == b_ref[...]).astype(jnp.float32)
    uw = jnp.dot(u_ref[...], wd_ref[...], preferred_element_type=jnp.float32)
    a_ref[...] = (jnp.dot(x, wa_ref[...], preferred_element_type=jnp.float32)
                  + jnp.dot(oh, uw, preferred_element_type=jnp.float32))
    bt_ref[...] = jnp.dot(x, wb_ref[...], preferred_element_type=jnp.float32)


def _node_tables(x, batch2, u, waT, wbT, wdT):
    return pl.pallas_call(
        _k0_body,
        grid=(NBN,),
        in_specs=[
            pl.BlockSpec((BLKN, F), lambda i: (i, 0)),
            pl.BlockSpec((BLKN, 1), lambda i: (i, 0)),
            pl.BlockSpec((G, F), lambda i: (0, 0)),
            pl.BlockSpec((F, F), lambda i: (0, 0)),
            pl.BlockSpec((F, F), lambda i: (0, 0)),
            pl.BlockSpec((F, F), lambda i: (0, 0)),
        ],
        out_specs=[pl.BlockSpec((BLKN, F), lambda i: (i, 0)),
                   pl.BlockSpec((BLKN, F), lambda i: (i, 0))],
        out_shape=[jax.ShapeDtypeStruct((N, F), jnp.float32),
                   jax.ShapeDtypeStruct((N, F), jnp.float32)],
    )(x, batch2, u, waT, wbT, wdT)


C1 = 1000
NCH1 = EW // C1
NSUB1 = C1 // SUB


def _k1_body(a_hbm, b_hbm, row_hbm, col_hbm, hp_hbm, c_hbm,
             idxr0, idxc0, ga0, gb0, ht0,
             idxr1, idxc1, ga1, gb1, ht1,
             ones, zflat,
             sema0, semb0, semc0, semw0,
             sema1, semb1, semc1, semw1, c_sh):
    scid = lax.axis_index("c")
    sid = lax.axis_index("s")
    wid = sid * NC + scid
    base = wid * EW
    iota16 = jax.lax.broadcasted_iota(jnp.int32, (F,), 0)
    bufs = ((idxr0, idxc0, ga0, gb0, ht0, sema0, semb0, semc0, semw0),
            (idxr1, idxc1, ga1, gb1, ht1, sema1, semb1, semc1, semw1))

    @plsc.parallel_loop(0, 128 // F, unroll=8)
    def _ofill(i):
        ones[pl.ds(i * F, F)] = jnp.full((F,), 1.0, jnp.float32)

    @plsc.parallel_loop(0, 1024 // F, unroll=8)
    def _zcfill(i):
        zflat[pl.ds(i * F, F)] = jnp.zeros((F,), jnp.float32)

    cstart = (sid * NSL) // 8 * 8
    for z in range(6):
        pltpu.sync_copy(zflat.at[pl.ds(0, 1000)],
                        c_sh.at[pl.ds(cstart + z * 1000, 1000)])
    pltpu.sync_copy(zflat.at[pl.ds(0, 256)], c_sh.at[pl.ds(cstart + 6000, 256)])
    plsc.subcore_barrier()

    def fire(buf, off):
        idxr, idxc, ga, gb, _, sema, semb, semc, _ = buf
        osub = pl.multiple_of(off // SUB, 8)
        pltpu.sync_copy(row_hbm.at[pl.ds(osub, NSUB1)], idxr)
        pltpu.sync_copy(col_hbm.at[pl.ds(osub, NSUB1)], idxc)
        for j in range(NSUB1):
            pltpu.async_copy(a_hbm.at[idxr.at[j]],
                             ga.at[pl.ds(j * SUB, SUB)], sema)
            pltpu.async_copy(b_hbm.at[idxc.at[j]],
                             gb.at[pl.ds(j * SUB, SUB)], semb)
            pltpu.async_copy(ones.at[pl.ds(0, SUB)], c_sh.at[idxc.at[j]],
                             semc, add=True)

    def drain_gathers(buf):
        _, _, ga, gb, _, sema, semb, semc, _ = buf
        pltpu.make_async_copy(a_hbm.at[pl.ds(0, C1)], ga, sema).wait()
        pltpu.make_async_copy(b_hbm.at[pl.ds(0, C1)], gb, semb).wait()
        pltpu.make_async_copy(c_hbm.at[0, pl.ds(0, C1)],
                              zflat.at[pl.ds(0, C1)], semc).wait()

    def compute(buf, off, first):
        _, _, ga, gb, htT, _, _, _, semw = buf

        @pl.when(jnp.logical_not(first))
        def _():
            pltpu.make_async_copy(htT, hp_hbm.at[:, pl.ds(0, C1)], semw).wait()

        @plsc.parallel_loop(0, C1, unroll=8)
        def _row(i):
            val = ga[i] + gb[i]
            plsc.store_scatter(htT, [iota16, jnp.full((F,), i, jnp.int32)],
                               val)

        pltpu.async_copy(htT, hp_hbm.at[:, pl.ds(off, C1)], semw)

    fire(bufs[0], base)

    @pl.loop(0, NCH1, step=2)
    def _chunk(ci):
        off = base + ci * C1
        fire(bufs[1], off + C1)
        drain_gathers(bufs[0])
        compute(bufs[0], off, ci == 0)

        @pl.when(ci + 2 < NCH1)
        def _():
            fire(bufs[0], off + 2 * C1)

        drain_gathers(bufs[1])
        compute(bufs[1], off + C1, ci == 0)

    for buf in bufs:
        htT, semw = buf[4], buf[8]
        pltpu.make_async_copy(htT, hp_hbm.at[:, pl.ds(0, C1)], semw).wait()
    plsc.subcore_barrier()
    pltpu.sync_copy(c_sh.at[pl.ds(cstart, 6256)],
                    c_hbm.at[scid, pl.ds(cstart, 6256)])


def _gather_hp(A, B, row3, col3):
    mesh = plsc.VectorSubcoreMesh(core_axis_name="c", subcore_axis_name="s")
    buf = [
        pltpu.VMEM((NSUB1, SUB), jnp.int32),
        pltpu.VMEM((NSUB1, SUB), jnp.int32),
        pltpu.VMEM((C1, F), jnp.float32),
        pltpu.VMEM((C1, F), jnp.float32),
        pltpu.VMEM((F, C1), jnp.float32),
    ]
    sems = [pltpu.SemaphoreType.DMA] * 4
    return pl.kernel(
        _k1_body,
        out_type=(jax.ShapeDtypeStruct((F, E), jnp.float32),
                  jax.ShapeDtypeStruct((NC, N), jnp.float32)),
        mesh=mesh,
        compiler_params=pltpu.CompilerParams(use_tc_tiling_on_sc=False, needs_layout_passes=False),
        scratch_types=buf + buf + [
            pltpu.VMEM((128,), jnp.float32),
            pltpu.VMEM((1024,), jnp.float32),
        ] + sems + sems + [
            pltpu.VMEM_SHARED((N,), jnp.float32),
        ],
    )(A, B, row3, col3)


EC = E // 128
CB = 200
NBC = EC // CB


def _k2a_body(hp_ref, ea_ref, wc_ref, h_ref, sum_ref, sq_ref):
    i = pl.program_id(0)

    @pl.when(i == 0)
    def _():
        sum_ref[...] = jnp.zeros_like(sum_ref)
        sq_ref[...] = jnp.zeros_like(sq_ref)

    wc = wc_ref[...]
    for c in range(CB):
        h = hp_ref[:, c, :] + jnp.dot(wc, ea_ref[:, c, :],
                                      preferred_element_type=jnp.float32)
        h_ref[:, c, :] = h
        sum_ref[...] += h
        sq_ref[...] += h * h


def _edge_h(hp3, ea3, wc):
    return pl.pallas_call(
        _k2a_body,
        grid=(NBC,),
        in_specs=[
            pl.BlockSpec((F, CB, 128), lambda i: (0, i, 0)),
            pl.BlockSpec((F, CB, 128), lambda i: (0, i, 0)),
            pl.BlockSpec((F, F), lambda i: (0, 0)),
        ],
        out_specs=[pl.BlockSpec((F, CB, 128), lambda i: (0, i, 0)),
                   pl.BlockSpec((F, 128), lambda i: (0, 0)),
                   pl.BlockSpec((F, 128), lambda i: (0, 0))],
        out_shape=[jax.ShapeDtypeStruct((F, EC, 128), jnp.float32),
                   jax.ShapeDtypeStruct((F, 128), jnp.float32),
                   jax.ShapeDtypeStruct((F, 128), jnp.float32)],
    )(hp3, ea3, wc)


def _k2b_body(h_ref, s_ref, t_ref, w2_ref, b2_ref, eo_ref):
    w2 = w2_ref[...]
    s = s_ref[...]
    t = t_ref[...]
    b2 = b2_ref[...]
    for c in range(CB):
        r = jnp.maximum(h_ref[:, c, :] * s + t, 0.0)
        eo_ref[:, c, :] = jnp.dot(w2, r,
                                  preferred_element_type=jnp.float32) + b2


def _edge_out(h3, scol, tcol, w2, b2col):
    return pl.pallas_call(
        _k2b_body,
        grid=(NBC,),
        in_specs=[
            pl.BlockSpec((F, CB, 128), lambda i: (0, i, 0)),
            pl.BlockSpec((F, 1), lambda i: (0, 0)),
            pl.BlockSpec((F, 1), lambda i: (0, 0)),
            pl.BlockSpec((F, F), lambda i: (0, 0)),
            pl.BlockSpec((F, 1), lambda i: (0, 0)),
        ],
        out_specs=pl.BlockSpec((F, CB, 128), lambda i: (0, i, 0)),
        out_shape=jax.ShapeDtypeStruct((F, EC, 128), jnp.float32),
    )(h3, scol, tcol, w2, b2col)


C3 = 1000
H3 = 500
NCH3 = EW // C3
NSUB3 = H3 // SUB


def _k3_body(h_hbm, col_hbm, s_hbm, t_hbm, sout_hbm,
             idxc, hv, hvT, sv_v, tv_v, s_sh):
    scid = lax.axis_index("c")
    sid = lax.axis_index("s")
    base = (scid * NS + sid) * EW
    iota16 = jax.lax.broadcasted_iota(jnp.int32, (F,), 0)

    pltpu.sync_copy(s_hbm, sv_v)
    pltpu.sync_copy(t_hbm, tv_v)
    sv = sv_v[...]
    tv = tv_v[...]

    @plsc.parallel_loop(0, H3, unroll=8)
    def _zfill(i):
        hv[i] = jnp.zeros((F,), jnp.float32)

    for z in range(13):
        zo = min(z * H3, NSL - H3)
        pltpu.sync_copy(hv, s_sh.at[pl.ds(sid * NSL + zo, H3)])
    plsc.subcore_barrier()

    @pl.loop(0, NCH3)
    def _chunk(ci):
        off = base + ci * C3
        pltpu.sync_copy(h_hbm.at[:, pl.ds(off, C3)], hvT)
        pltpu.sync_copy(col_hbm.at[pl.ds(pl.multiple_of(off // SUB, 8), C3 // SUB)],
                        idxc)
        for h2 in range(2):

            @plsc.parallel_loop(0, H3, unroll=8)
            def _row(i):
                hcol = plsc.load_gather(
                    hvT, [iota16, jnp.full((F,), h2 * H3 + i, jnp.int32)])
                hv[i] = jnp.maximum(hcol * sv + tv, 0.0)

            for j in range(NSUB3):
                pltpu.sync_copy(hv.at[pl.ds(j * SUB, SUB)],
                                s_sh.at[idxc.at[h2 * NSUB3 + j]], add=True)

    plsc.subcore_barrier()
    pltpu.sync_copy(s_sh.at[pl.ds(sid * NSL, NSL)],
                    sout_hbm.at[scid, pl.ds(sid * NSL, NSL)])


def _scatter_r(h, col3, s16, t16):
    mesh = plsc.VectorSubcoreMesh(core_axis_name="c", subcore_axis_name="s")
    return pl.kernel(
        _k3_body,
        out_type=jax.ShapeDtypeStruct((NC, N, F), jnp.float32),
        mesh=mesh,
        compiler_params=pltpu.CompilerParams(use_tc_tiling_on_sc=False, needs_layout_passes=False),
        scratch_types=[
            pltpu.VMEM((C3 // SUB, SUB), jnp.int32),
            pltpu.VMEM((H3, F), jnp.float32),
            pltpu.VMEM((F, C3), jnp.float32),
            pltpu.VMEM((F,), jnp.float32),
            pltpu.VMEM((F,), jnp.float32),
            pltpu.VMEM_SHARED((N, F), jnp.float32),
        ],
    )(h, col3, s16, t16)


def _k4_body(x_ref, s0_ref, s1_ref, c0_ref, c1_ref, b2_ref, bt_ref, u_ref,
             mats_ref, vecs_ref,
             xo_ref, uo_ref,
             nsum, nsq, xa, sg, ecg, ncnt):
    i = pl.program_id(0)
    nwaT = mats_ref[0:16, :]
    nwbT = mats_ref[16:32, :]
    nw2T = mats_ref[32:48, :]
    ew2T = mats_ref[48:64, :]
    gwaT = mats_ref[64:80, :]
    gwbT = mats_ref[80:96, :]
    gwcT = mats_ref[96:112, :]
    n_b1 = vecs_ref[0:1, :]
    n_b2 = vecs_ref[1:2, :]
    n_g1 = vecs_ref[2:3, :]
    n_be1 = vecs_ref[3:4, :]
    e_b2 = vecs_ref[4:5, :]
    g_b1 = vecs_ref[5:6, :]
    g_g1 = vecs_ref[6:7, :]
    g_be1 = vecs_ref[7:8, :]

    bT = bt_ref[0]
    ohT = (lax.broadcasted_iota(jnp.int32, (G, BLKN), 0) == bT).astype(jnp.float32)

    def node_hidden():
        s_blk = s0_ref[...] + s1_ref[...]
        c_blk = jnp.broadcast_to(c0_ref[...] + c1_ref[...], (BLKN, F))
        e_aggr = ((jnp.dot(s_blk, ew2T, preferred_element_type=jnp.float32)
                   + c_blk * e_b2) / jnp.maximum(c_blk, 1.0))
        oh = (lax.broadcasted_iota(jnp.int32, (BLKN, G), 1)
              == b2_ref[...]).astype(jnp.float32)
        hn = (jnp.dot(x_ref[...], nwaT, preferred_element_type=jnp.float32)
              + jnp.dot(e_aggr, nwbT, preferred_element_type=jnp.float32)
              + jnp.dot(oh, jnp.dot(u_ref[...], mats_ref[112:128, :],
                                    preferred_element_type=jnp.float32),
                        preferred_element_type=jnp.float32)
              + n_b1)
        return hn, s_blk, c_blk

    @pl.when(i == 0)
    def _():
        nsum[...] = jnp.zeros_like(nsum)
        nsq[...] = jnp.zeros_like(nsq)
        xa[...] = jnp.zeros_like(xa)
        sg[...] = jnp.zeros_like(sg)
        ecg[...] = jnp.zeros_like(ecg)
        ncnt[...] = jnp.zeros_like(ncnt)

    @pl.when(i < NBN)
    def _phase_a():
        hn, s_blk, c_blk = node_hidden()
        nsum[...] += jnp.sum(hn, axis=0, keepdims=True)
        nsq[...] += jnp.sum(hn * hn, axis=0, keepdims=True)
        sg[...] += jnp.dot(ohT, s_blk, preferred_element_type=jnp.float32)
        ecg[...] += jnp.dot(ohT, c_blk, preferred_element_type=jnp.float32)
        ncnt[...] += jnp.dot(ohT, jnp.ones((BLKN, F), jnp.float32),
                             preferred_element_type=jnp.float32)

    @pl.when(i >= NBN)
    def _phase_b():
        hn, _, _ = node_hidden()
        mn = nsum[...] / N
        vr = nsq[...] / N - mn * mn
        sn = n_g1 * lax.rsqrt(vr + EPS)
        tn = n_be1 - mn * sn
        xo = jnp.dot(jnp.maximum(hn * sn + tn, 0.0), nw2T,
                     preferred_element_type=jnp.float32) + n_b2
        xo_ref[...] = xo
        w = jnp.where(i < 2 * NBN, 1.0, 0.0).astype(jnp.float32)
        xa[...] += w * jnp.dot(ohT, xo, preferred_element_type=jnp.float32)

    @pl.when(i == 2 * NBN)
    def _phase_g():
        x_aggr = xa[...] / jnp.maximum(ncnt[...], 1.0)
        e_aggr_g = ((jnp.dot(sg[...], ew2T, preferred_element_type=jnp.float32)
                     + ecg[...] * e_b2) / jnp.maximum(ecg[...], 1.0))
        go = (jnp.dot(u_ref[...], gwaT, preferred_element_type=jnp.float32)
              + jnp.dot(x_aggr, gwbT, preferred_element_type=jnp.float32)
              + jnp.dot(e_aggr_g, gwcT, preferred_element_type=jnp.float32)
              + g_b1)
        mg = jnp.sum(go, axis=0, keepdims=True) / G
        vg = jnp.sum(go * go, axis=0, keepdims=True) / G - mg * mg
        uo_ref[...] = jnp.maximum(
            (go - mg) * lax.rsqrt(vg + EPS) * g_g1 + g_be1, 0.0)


def _node_global(x, S, c2, batch2, batchT, u, mats, vecs):
    S0 = S[0]
    S1 = S[1]
    c0 = c2[0].reshape(N, 1)
    c1 = c2[1].reshape(N, 1)
    nb = lambda i: (i % NBN, 0)
    return pl.pallas_call(
        _k4_body,
        grid=(2 * NBN + 1,),
        in_specs=[
            pl.BlockSpec((BLKN, F), nb),
            pl.BlockSpec((BLKN, F), nb),
            pl.BlockSpec((BLKN, F), nb),
            pl.BlockSpec((BLKN, 1), nb),
            pl.BlockSpec((BLKN, 1), nb),
            pl.BlockSpec((BLKN, 1), nb),
            pl.BlockSpec((1, 1, BLKN), lambda i: (i % NBN, 0, 0)),
            pl.BlockSpec((G, F), lambda i: (0, 0)),
            pl.BlockSpec((128, F), lambda i: (0, 0)),
            pl.BlockSpec((8, F), lambda i: (0, 0)),
        ],
        out_specs=[pl.BlockSpec((BLKN, F), nb),
                   pl.BlockSpec((G, F), lambda i: (0, 0))],
        out_shape=[jax.ShapeDtypeStruct((N, F), jnp.float32),
                   jax.ShapeDtypeStruct((G, F), jnp.float32)],
        scratch_shapes=[
            pltpu.VMEM((1, F), jnp.float32),
            pltpu.VMEM((1, F), jnp.float32),
            pltpu.VMEM((G, F), jnp.float32),
            pltpu.VMEM((G, F), jnp.float32),
            pltpu.VMEM((G, F), jnp.float32),
            pltpu.VMEM((G, F), jnp.float32),
        ],
    )(x, S0, S1, c0, c1, batch2, batchT, u, mats, vecs)


def kernel(x, edge_index, edge_attr, u, batch,
           e_W1, e_b1, e_g1, e_be1, e_W2, e_b2,
           n_W1, n_b1, n_g1, n_be1, n_W2, n_b2,
           g_W1, g_b1, g_g1, g_be1):
    row = edge_index[0].astype(jnp.int32)
    col = edge_index[1].astype(jnp.int32)
    batch = batch.astype(jnp.int32)

    eWaT = e_W1[:, 0:16].T
    eWbT = e_W1[:, 16:32].T
    eWcT = e_W1[:, 32:48].T
    eWdT = e_W1[:, 48:64].T

    batch2 = batch.reshape(N, 1)
    batchT = batch.reshape(NBN, 1, BLKN)
    row3 = row.reshape(E // SUB, SUB)
    col3 = col.reshape(E // SUB, SUB)

    A, B = _node_tables(x, batch2, u, eWaT, eWbT, eWdT)

    hp, c2 = _gather_hp(A, B, row3, col3)

    ea3 = edge_attr.T.reshape(F, EC, 128)
    hp3 = hp.reshape(F, EC, 128)
    h3, hsum, hsq = _edge_h(hp3, ea3, e_W1[:, 32:48])

    hsum16 = jnp.sum(hsum, axis=1)
    hsq16 = jnp.sum(hsq, axis=1)
    mean = hsum16 / E
    var = hsq16 / E - mean * mean
    s16 = e_g1 * lax.rsqrt(var + EPS)
    t16 = e_be1 - mean * s16

    eo3 = _edge_out(h3, s16.reshape(F, 1), t16.reshape(F, 1), e_W2,
                    e_b2.reshape(F, 1))
    edge_out = eo3.reshape(F, E).T

    S = _scatter_r(h3.reshape(F, E), col3, s16, t16)

    mats = jnp.concatenate([
        n_W1[:, 0:16].T, n_W1[:, 16:32].T, n_W2.T, e_W2.T,
        g_W1[:, 0:16].T, g_W1[:, 16:32].T, g_W1[:, 32:48].T,
        n_W1[:, 32:48].T,
    ], axis=0)
    vecs = jnp.stack([n_b1, n_b2, n_g1, n_be1, e_b2, g_b1, g_g1, g_be1])
    x_out, u_out = _node_global(x, S, c2, batch2, batchT, u, mats, vecs)

    return (x_out, edge_out, u_out)


# speedup vs baseline: 2.8742x; 1.0162x over previous
import functools

import jax
import jax.numpy as jnp
from jax import lax
from jax.experimental import pallas as pl
from jax.experimental.pallas import tpu as pltpu
from jax.experimental.pallas import tpu_sc as plsc

N = 100000
E = 3200000
F = 16
G = 256
EPS = 1e-5

NC = 2
NS = 16
NW = NC * NS
EW = E // NW
C = 2000
NCH = EW // C
SUB = 125
NSUB = C // SUB
NSL = N // NS

E8 = E // 8
BLKE = 2000
NBE = E8 // BLKE
BLKN = 4000
NBN = N // BLKN


def _bd(w):
    return jnp.kron(jnp.eye(8, dtype=w.dtype), w)


def _k0_body(x_ref, b_ref, u_ref, wa_ref, wb_ref, wd_ref, a_ref, bt_ref):
    x = x_ref[...]
    oh = (lax.broadcasted_iota(jnp.int32, (BLKN, G), 1) == b_ref[...]).astype(jnp.float32)
    uw = jnp.dot(u_ref[...], wd_ref[...], preferred_element_type=jnp.float32)
    a_ref[...] = (jnp.dot(x, wa_ref[...], preferred_element_type=jnp.float32)
                  + jnp.dot(oh, uw, preferred_element_type=jnp.float32))
    bt_ref[...] = jnp.dot(x, wb_ref[...], preferred_element_type=jnp.float32)


def _node_tables(x, batch2, u, waT, wbT, wdT):
    return pl.pallas_call(
        _k0_body,
        grid=(NBN,),
        in_specs=[
            pl.BlockSpec((BLKN, F), lambda i: (i, 0)),
            pl.BlockSpec((BLKN, 1), lambda i: (i, 0)),
            pl.BlockSpec((G, F), lambda i: (0, 0)),
            pl.BlockSpec((F, F), lambda i: (0, 0)),
            pl.BlockSpec((F, F), lambda i: (0, 0)),
            pl.BlockSpec((F, F), lambda i: (0, 0)),
        ],
        out_specs=[pl.BlockSpec((BLKN, F), lambda i: (i, 0)),
                   pl.BlockSpec((BLKN, F), lambda i: (i, 0))],
        out_shape=[jax.ShapeDtypeStruct((N, F), jnp.float32),
                   jax.ShapeDtypeStruct((N, F), jnp.float32)],
    )(x, batch2, u, waT, wbT, wdT)


C1 = 1000
NCH1 = EW // C1
NSUB1 = C1 // SUB


def _k1_body(a_hbm, b_hbm, row_hbm, col_hbm, hp_hbm, c_hbm,
             idxr0, idxc0, ga0, gb0, ht0,
             idxr1, idxc1, ga1, gb1, ht1,
             ones, zflat,
             sema0, semb0, semc0, semw0,
             sema1, semb1, semc1, semw1, c_sh):
    scid = lax.axis_index("c")
    sid = lax.axis_index("s")
    wid = sid * NC + scid
    base = wid * EW
    iota16 = jax.lax.broadcasted_iota(jnp.int32, (F,), 0)
    bufs = ((idxr0, idxc0, ga0, gb0, ht0, sema0, semb0, semc0, semw0),
            (idxr1, idxc1, ga1, gb1, ht1, sema1, semb1, semc1, semw1))

    @plsc.parallel_loop(0, 128 // F, unroll=8)
    def _ofill(i):
        ones[pl.ds(i * F, F)] = jnp.full((F,), 1.0, jnp.float32)

    @plsc.parallel_loop(0, 1024 // F, unroll=8)
    def _zcfill(i):
        zflat[pl.ds(i * F, F)] = jnp.zeros((F,), jnp.float32)

    cstart = (sid * NSL) // 8 * 8
    for z in range(6):
        pltpu.sync_copy(zflat.at[pl.ds(0, 1000)],
                        c_sh.at[pl.ds(cstart + z * 1000, 1000)])
    pltpu.sync_copy(zflat.at[pl.ds(0, 256)], c_sh.at[pl.ds(cstart + 6000, 256)])
    plsc.subcore_barrier()

    def fire(buf, off):
        idxr, idxc, ga, gb, _, sema, semb, semc, _ = buf
        osub = pl.multiple_of(off // SUB, 8)
        pltpu.sync_copy(row_hbm.at[pl.ds(osub, NSUB1)], idxr)
        pltpu.sync_copy(col_hbm.at[pl.ds(osub, NSUB1)], idxc)
        for j in range(NSUB1):
            pltpu.async_copy(a_hbm.at[idxr.at[j]],
                             ga.at[pl.ds(j * SUB, SUB)], sema)
            pltpu.async_copy(b_hbm.at[idxc.at[j]],
                             gb.at[pl.ds(j * SUB, SUB)], semb)
            pltpu.async_copy(ones.at[pl.ds(0, SUB)], c_sh.at[idxc.at[j]],
                             semc, add=True)

    def drain_gathers(buf):
        _, _, ga, gb, _, sema, semb, semc, _ = buf
        pltpu.make_async_copy(a_hbm.at[pl.ds(0, C1)], ga, sema).wait()
        pltpu.make_async_copy(b_hbm.at[pl.ds(0, C1)], gb, semb).wait()
        pltpu.make_async_copy(c_hbm.at[0, pl.ds(0, C1)],
                              zflat.at[pl.ds(0, C1)], semc).wait()

    def compute(buf, off, first):
        _, _, ga, gb, htT, _, _, _, semw = buf

        @pl.when(jnp.logical_not(first))
        def _():
            pltpu.make_async_copy(htT, hp_hbm.at[:, pl.ds(0, C1)], semw).wait()

        @plsc.parallel_loop(0, C1, unroll=8)
        def _row(i):
            val = ga[i] + gb[i]
            plsc.store_scatter(htT, [iota16, jnp.full((F,), i, jnp.int32)],
                               val)

        pltpu.async_copy(htT, hp_hbm.at[:, pl.ds(off, C1)], semw)

    fire(bufs[0], base)

    @pl.loop(0, NCH1, step=2)
    def _chunk(ci):
        off = base + ci * C1
        fire(bufs[1], off + C1)
        drain_gathers(bufs[0])
        compute(bufs[0], off, ci == 0)

        @pl.when(ci + 2 < NCH1)
        def _():
            fire(bufs[0], off + 2 * C1)

        drain_gathers(bufs[1])
        compute(bufs[1], off + C1, ci == 0)

    for buf in bufs:
        htT, semw = buf[4], buf[8]
        pltpu.make_async_copy(htT, hp_hbm.at[:, pl.ds(0, C1)], semw).wait()
    plsc.subcore_barrier()
    pltpu.sync_copy(c_sh.at[pl.ds(cstart, 6256)],
                    c_hbm.at[scid, pl.ds(cstart, 6256)])


def _gather_hp(A, B, row3, col3):
    mesh = plsc.VectorSubcoreMesh(core_axis_name="c", subcore_axis_name="s")
    buf = [
        pltpu.VMEM((NSUB1, SUB), jnp.int32),
        pltpu.VMEM((NSUB1, SUB), jnp.int32),
        pltpu.VMEM((C1, F), jnp.float32),
        pltpu.VMEM((C1, F), jnp.float32),
        pltpu.VMEM((F, C1), jnp.float32),
    ]
    sems = [pltpu.SemaphoreType.DMA] * 4
    return pl.kernel(
        _k1_body,
        out_type=(jax.ShapeDtypeStruct((F, E), jnp.float32),
                  jax.ShapeDtypeStruct((NC, N), jnp.float32)),
        mesh=mesh,
        compiler_params=pltpu.CompilerParams(use_tc_tiling_on_sc=False, needs_layout_passes=False),
        scratch_types=buf + buf + [
            pltpu.VMEM((128,), jnp.float32),
            pltpu.VMEM((1024,), jnp.float32),
        ] + sems + sems + [
            pltpu.VMEM_SHARED((N,), jnp.float32),
        ],
    )(A, B, row3, col3)


EC = E // 128
CB = 200
NBC = EC // CB


def _k2a_body(hp_ref, ea_ref, wc_ref, h_ref, sum_ref, sq_ref):
    i = pl.program_id(0)

    @pl.when(i == 0)
    def _():
        sum_ref[...] = jnp.zeros_like(sum_ref)
        sq_ref[...] = jnp.zeros_like(sq_ref)

    wc = wc_ref[...]
    for c in range(CB):
        h = hp_ref[:, c, :] + jnp.dot(wc, ea_ref[:, c, :],
                                      preferred_element_type=jnp.float32)
        h_ref[:, c, :] = h
        sum_ref[...] += h
        sq_ref[...] += h * h


def _edge_h(hp3, ea3, wc):
    return pl.pallas_call(
        _k2a_body,
        grid=(NBC,),
        in_specs=[
            pl.BlockSpec((F, CB, 128), lambda i: (0, i, 0)),
            pl.BlockSpec((F, CB, 128), lambda i: (0, i, 0)),
            pl.BlockSpec((F, F), lambda i: (0, 0)),
        ],
        out_specs=[pl.BlockSpec((F, CB, 128), lambda i: (0, i, 0)),
                   pl.BlockSpec((F, 128), lambda i: (0, 0)),
                   pl.BlockSpec((F, 128), lambda i: (0, 0))],
        out_shape=[jax.ShapeDtypeStruct((F, EC, 128), jnp.float32),
                   jax.ShapeDtypeStruct((F, 128), jnp.float32),
                   jax.ShapeDtypeStruct((F, 128), jnp.float32)],
    )(hp3, ea3, wc)


def _k2b_body(h_ref, s_ref, t_ref, w2_ref, b2_ref, eo_ref):
    w2 = w2_ref[...]
    s = s_ref[...]
    t = t_ref[...]
    b2 = b2_ref[...]
    for c in range(CB):
        r = jnp.maximum(h_ref[:, c, :] * s + t, 0.0)
        eo_ref[:, c, :] = jnp.dot(w2, r,
                                  preferred_element_type=jnp.float32) + b2


def _edge_out(h3, scol, tcol, w2, b2col):
    return pl.pallas_call(
        _k2b_body,
        grid=(NBC,),
        in_specs=[
            pl.BlockSpec((F, CB, 128), lambda i: (0, i, 0)),
            pl.BlockSpec((F, 1), lambda i: (0, 0)),
            pl.BlockSpec((F, 1), lambda i: (0, 0)),
            pl.BlockSpec((F, F), lambda i: (0, 0)),
            pl.BlockSpec((F, 1), lambda i: (0, 0)),
        ],
        out_specs=pl.BlockSpec((F, CB, 128), lambda i: (0, i, 0)),
        out_shape=jax.ShapeDtypeStruct((F, EC, 128), jnp.float32),
    )(h3, scol, tcol, w2, b2col)


C3 = 1000
H3 = 500
NCH3 = EW // C3
NSUB3 = H3 // SUB


def _k3_body(h_hbm, col_hbm, s_hbm, t_hbm, sout_hbm,
             idxc, hv, hvT, sv_v, tv_v, sems, s_sh):
    scid = lax.axis_index("c")
    sid = lax.axis_index("s")
    base = (scid * NS + sid) * EW
    iota16 = jax.lax.broadcasted_iota(jnp.int32, (F,), 0)

    pltpu.sync_copy(s_hbm, sv_v)
    pltpu.sync_copy(t_hbm, tv_v)
    sv = sv_v[...]
    tv = tv_v[...]

    @plsc.parallel_loop(0, H3, unroll=8)
    def _zfill(i):
        hv[i] = jnp.zeros((F,), jnp.float32)

    for z in range(13):
        zo = min(z * H3, NSL - H3)
        pltpu.sync_copy(hv, s_sh.at[pl.ds(sid * NSL + zo, H3)])
    plsc.subcore_barrier()

    @pl.loop(0, NCH3)
    def _chunk(ci):
        off = base + ci * C3
        pltpu.sync_copy(h_hbm.at[:, pl.ds(off, C3)], hvT)
        pltpu.sync_copy(col_hbm.at[pl.ds(pl.multiple_of(off // SUB, 8), C3 // SUB)],
                        idxc)
        for h2 in range(2):

            @plsc.parallel_loop(0, H3, unroll=8)
            def _row(i):
                hcol = plsc.load_gather(
                    hvT, [iota16, jnp.full((F,), h2 * H3 + i, jnp.int32)])
                hv[i] = jnp.maximum(hcol * sv + tv, 0.0)

            for j in range(NSUB3):
                pltpu.async_copy(hv.at[pl.ds(j * SUB, SUB)],
                                 s_sh.at[idxc.at[h2 * NSUB3 + j]], sems,
                                 add=True)
            pltpu.make_async_copy(sout_hbm.at[0, pl.ds(0, H3)], hv,
                                  sems).wait()

    plsc.subcore_barrier()
    pltpu.sync_copy(s_sh.at[pl.ds(sid * NSL, NSL)],
                    sout_hbm.at[scid, pl.ds(sid * NSL, NSL)])


def _scatter_r(h, col3, s16, t16):
    mesh = plsc.VectorSubcoreMesh(core_axis_name="c", subcore_axis_name="s")
    return pl.kernel(
        _k3_body,
        out_type=jax.ShapeDtypeStruct((NC, N, F), jnp.float32),
        mesh=mesh,
        compiler_params=pltpu.CompilerParams(use_tc_tiling_on_sc=False, needs_layout_passes=False),
        scratch_types=[
            pltpu.VMEM((C3 // SUB, SUB), jnp.int32),
            pltpu.VMEM((H3, F), jnp.float32),
            pltpu.VMEM((F, C3), jnp.float32),
            pltpu.VMEM((F,), jnp.float32),
            pltpu.VMEM((F,), jnp.float32),
            pltpu.SemaphoreType.DMA,
            pltpu.VMEM_SHARED((N, F), jnp.float32),
        ],
    )(h, col3, s16, t16)


def _k4_body(x_ref, s0_ref, s1_ref, c0_ref, c1_ref, b2_ref, bt_ref, u_ref,
             mats_ref, vecs_ref,
             xo_ref, uo_ref,
             nsum, nsq, xa, sg, ecg, ncnt):
    i = pl.program_id(0)
    nwaT = mats_ref[0:16, :]
    nwbT = mats_ref[16:32, :]
    nw2T = mats_ref[32:48, :]
    ew2T = mats_ref[48:64, :]
    gwaT = mats_ref[64:80, :]
    gwbT = mats_ref[80:96, :]
    gwcT = mats_ref[96:112, :]
    n_b1 = vecs_ref[0:1, :]
    n_b2 = vecs_ref[1:2, :]
    n_g1 = vecs_ref[2:3, :]
    n_be1 = vecs_ref[3:4, :]
    e_b2 = vecs_ref[4:5, :]
    g_b1 = vecs_ref[5:6, :]
    g_g1 = vecs_ref[6:7, :]
    g_be1 = vecs_ref[7:8, :]

    bT = bt_ref[0]
    ohT = (lax.broadcasted_iota(jnp.int32, (G, BLKN), 0) == bT).astype(jnp.float32)

    def node_hidden():
        s_blk = s0_ref[...] + s1_ref[...]
        c_blk = jnp.broadcast_to(c0_ref[...] + c1_ref[...], (BLKN, F))
        e_aggr = ((jnp.dot(s_blk, ew2T, preferred_element_type=jnp.float32)
                   + c_blk * e_b2) / jnp.maximum(c_blk, 1.0))
        oh = (lax.broadcasted_iota(jnp.int32, (BLKN, G), 1)
              == b2_ref[...]).astype(jnp.float32)
        hn = (jnp.dot(x_ref[...], nwaT, preferred_element_type=jnp.float32)
              + jnp.dot(e_aggr, nwbT, preferred_element_type=jnp.float32)
              + jnp.dot(oh, jnp.dot(u_ref[...], mats_ref[112:128, :],
                                    preferred_element_type=jnp.float32),
                        preferred_element_type=jnp.float32)
              + n_b1)
        return hn, s_blk, c_blk

    @pl.when(i == 0)
    def _():
        nsum[...] = jnp.zeros_like(nsum)
        nsq[...] = jnp.zeros_like(nsq)
        xa[...] = jnp.zeros_like(xa)
        sg[...] = jnp.zeros_like(sg)
        ecg[...] = jnp.zeros_like(ecg)
        ncnt[...] = jnp.zeros_like(ncnt)

    @pl.when(i < NBN)
    def _phase_a():
        hn, s_blk, c_blk = node_hidden()
        nsum[...] += jnp.sum(hn, axis=0, keepdims=True)
        nsq[...] += jnp.sum(hn * hn, axis=0, keepdims=True)
        sg[...] += jnp.dot(ohT, s_blk, preferred_element_type=jnp.float32)
        ecg[...] += jnp.dot(ohT, c_blk, preferred_element_type=jnp.float32)
        ncnt[...] += jnp.dot(ohT, jnp.ones((BLKN, F), jnp.float32),
                             preferred_element_type=jnp.float32)

    @pl.when(i >= NBN)
    def _phase_b():
        hn, _, _ = node_hidden()
        mn = nsum[...] / N
        vr = nsq[...] / N - mn * mn
        sn = n_g1 * lax.rsqrt(vr + EPS)
        tn = n_be1 - mn * sn
        xo = jnp.dot(jnp.maximum(hn * sn + tn, 0.0), nw2T,
                     preferred_element_type=jnp.float32) + n_b2
        xo_ref[...] = xo
        w = jnp.where(i < 2 * NBN, 1.0, 0.0).astype(jnp.float32)
        xa[...] += w * jnp.dot(ohT, xo, preferred_element_type=jnp.float32)

    @pl.when(i == 2 * NBN)
    def _phase_g():
        x_aggr = xa[...] / jnp.maximum(ncnt[...], 1.0)
        e_aggr_g = ((jnp.dot(sg[...], ew2T, preferred_element_type=jnp.float32)
                     + ecg[...] * e_b2) / jnp.maximum(ecg[...], 1.0))
        go = (jnp.dot(u_ref[...], gwaT, preferred_element_type=jnp.float32)
              + jnp.dot(x_aggr, gwbT, preferred_element_type=jnp.float32)
              + jnp.dot(e_aggr_g, gwcT, preferred_element_type=jnp.float32)
              + g_b1)
        mg = jnp.sum(go, axis=0, keepdims=True) / G
        vg = jnp.sum(go * go, axis=0, keepdims=True) / G - mg * mg
        uo_ref[...] = jnp.maximum(
            (go - mg) * lax.rsqrt(vg + EPS) * g_g1 + g_be1, 0.0)


def _node_global(x, S, c2, batch2, batchT, u, mats, vecs):
    S0 = S[0]
    S1 = S[1]
    c0 = c2[0].reshape(N, 1)
    c1 = c2[1].reshape(N, 1)
    nb = lambda i: (i % NBN, 0)
    return pl.pallas_call(
        _k4_body,
        grid=(2 * NBN + 1,),
        in_specs=[
            pl.BlockSpec((BLKN, F), nb),
            pl.BlockSpec((BLKN, F), nb),
            pl.BlockSpec((BLKN, F), nb),
            pl.BlockSpec((BLKN, 1), nb),
            pl.BlockSpec((BLKN, 1), nb),
            pl.BlockSpec((BLKN, 1), nb),
            pl.BlockSpec((1, 1, BLKN), lambda i: (i % NBN, 0, 0)),
            pl.BlockSpec((G, F), lambda i: (0, 0)),
            pl.BlockSpec((128, F), lambda i: (0, 0)),
            pl.BlockSpec((8, F), lambda i: (0, 0)),
        ],
        out_specs=[pl.BlockSpec((BLKN, F), nb),
                   pl.BlockSpec((G, F), lambda i: (0, 0))],
        out_shape=[jax.ShapeDtypeStruct((N, F), jnp.float32),
                   jax.ShapeDtypeStruct((G, F), jnp.float32)],
        scratch_shapes=[
            pltpu.VMEM((1, F), jnp.float32),
            pltpu.VMEM((1, F), jnp.float32),
            pltpu.VMEM((G, F), jnp.float32),
            pltpu.VMEM((G, F), jnp.float32),
            pltpu.VMEM((G, F), jnp.float32),
            pltpu.VMEM((G, F), jnp.float32),
        ],
    )(x, S0, S1, c0, c1, batch2, batchT, u, mats, vecs)


def kernel(x, edge_index, edge_attr, u, batch,
           e_W1, e_b1, e_g1, e_be1, e_W2, e_b2,
           n_W1, n_b1, n_g1, n_be1, n_W2, n_b2,
           g_W1, g_b1, g_g1, g_be1):
    row = edge_index[0].astype(jnp.int32)
    col = edge_index[1].astype(jnp.int32)
    batch = batch.astype(jnp.int32)

    eWaT = e_W1[:, 0:16].T
    eWbT = e_W1[:, 16:32].T
    eWcT = e_W1[:, 32:48].T
    eWdT = e_W1[:, 48:64].T

    batch2 = batch.reshape(N, 1)
    batchT = batch.reshape(NBN, 1, BLKN)
    row3 = row.reshape(E // SUB, SUB)
    col3 = col.reshape(E // SUB, SUB)

    A, B = _node_tables(x, batch2, u, eWaT, eWbT, eWdT)

    hp, c2 = _gather_hp(A, B, row3, col3)

    ea3 = edge_attr.T.reshape(F, EC, 128)
    hp3 = hp.reshape(F, EC, 128)
    h3, hsum, hsq = _edge_h(hp3, ea3, e_W1[:, 32:48])

    hsum16 = jnp.sum(hsum, axis=1)
    hsq16 = jnp.sum(hsq, axis=1)
    mean = hsum16 / E
    var = hsq16 / E - mean * mean
    s16 = e_g1 * lax.rsqrt(var + EPS)
    t16 = e_be1 - mean * s16

    eo3 = _edge_out(h3, s16.reshape(F, 1), t16.reshape(F, 1), e_W2,
                    e_b2.reshape(F, 1))
    edge_out = eo3.reshape(F, E).T

    S = _scatter_r(h3.reshape(F, E), col3, s16, t16)

    mats = jnp.concatenate([
        n_W1[:, 0:16].T, n_W1[:, 16:32].T, n_W2.T, e_W2.T,
        g_W1[:, 0:16].T, g_W1[:, 16:32].T, g_W1[:, 32:48].T,
        n_W1[:, 32:48].T,
    ], axis=0)
    vecs = jnp.stack([n_b1, n_b2, n_g1, n_be1, e_b2, g_b1, g_g1, g_be1])
    x_out, u_out = _node_global(x, S, c2, batch2, batchT, u, mats, vecs)

    return (x_out, edge_out, u_out)


# speedup vs baseline: 2.9139x; 1.0138x over previous
import functools

import jax
import jax.numpy as jnp
from jax import lax
from jax.experimental import pallas as pl
from jax.experimental.pallas import tpu as pltpu
from jax.experimental.pallas import tpu_sc as plsc

N = 100000
E = 3200000
F = 16
G = 256
EPS = 1e-5

NC = 2
NS = 16
NW = NC * NS
EW = E // NW
C = 2000
NCH = EW // C
SUB = 125
NSUB = C // SUB
NSL = N // NS

E8 = E // 8
BLKE = 2000
NBE = E8 // BLKE
BLKN = 4000
NBN = N // BLKN


def _bd(w):
    return jnp.kron(jnp.eye(8, dtype=w.dtype), w)


def _k0_body(x_ref, b_ref, u_ref, wa_ref, wb_ref, wd_ref, a_ref, bt_ref):
    x = x_ref[...]
    oh = (lax.broadcasted_iota(jnp.int32, (BLKN, G), 1) == b_ref[...]).astype(jnp.float32)
    uw = jnp.dot(u_ref[...], wd_ref[...], preferred_element_type=jnp.float32)
    a_ref[...] = (jnp.dot(x, wa_ref[...], preferred_element_type=jnp.float32)
                  + jnp.dot(oh, uw, preferred_element_type=jnp.float32))
    bt_ref[...] = jnp.dot(x, wb_ref[...], preferred_element_type=jnp.float32)


def _node_tables(x, batch2, u, waT, wbT, wdT):
    return pl.pallas_call(
        _k0_body,
        grid=(NBN,),
        in_specs=[
            pl.BlockSpec((BLKN, F), lambda i: (i, 0)),
            pl.BlockSpec((BLKN, 1), lambda i: (i, 0)),
            pl.BlockSpec((G, F), lambda i: (0, 0)),
            pl.BlockSpec((F, F), lambda i: (0, 0)),
            pl.BlockSpec((F, F), lambda i: (0, 0)),
            pl.BlockSpec((F, F), lambda i: (0, 0)),
        ],
        out_specs=[pl.BlockSpec((BLKN, F), lambda i: (i, 0)),
                   pl.BlockSpec((BLKN, F), lambda i: (i, 0))],
        out_shape=[jax.ShapeDtypeStruct((N, F), jnp.float32),
                   jax.ShapeDtypeStruct((N, F), jnp.float32)],
    )(x, batch2, u, waT, wbT, wdT)


C1 = 1000
NCH1 = EW // C1
NSUB1 = C1 // SUB


def _k1_body(a_hbm, b_hbm, row_hbm, col_hbm, hp_hbm, c_hbm,
             idxr0, idxc0, ga0, gb0, ht0,
             idxr1, idxc1, ga1, gb1, ht1,
             ones, zflat,
             sema0, semb0, semc0, semw0,
             sema1, semb1, semc1, semw1, c_sh):
    scid = lax.axis_index("c")
    sid = lax.axis_index("s")
    wid = sid * NC + scid
    base = wid * EW
    iota16 = jax.lax.broadcasted_iota(jnp.int32, (F,), 0)
    bufs = ((idxr0, idxc0, ga0, gb0, ht0, sema0, semb0, semc0, semw0),
            (idxr1, idxc1, ga1, gb1, ht1, sema1, semb1, semc1, semw1))

    @plsc.parallel_loop(0, 128 // F, unroll=8)
    def _ofill(i):
        ones[pl.ds(i * F, F)] = jnp.full((F,), 1.0, jnp.float32)

    @plsc.parallel_loop(0, 1024 // F, unroll=8)
    def _zcfill(i):
        zflat[pl.ds(i * F, F)] = jnp.zeros((F,), jnp.float32)

    cstart = (sid * NSL) // 8 * 8
    for z in range(6):
        pltpu.sync_copy(zflat.at[pl.ds(0, 1000)],
                        c_sh.at[pl.ds(cstart + z * 1000, 1000)])
    pltpu.sync_copy(zflat.at[pl.ds(0, 256)], c_sh.at[pl.ds(cstart + 6000, 256)])
    plsc.subcore_barrier()

    def fire(buf, off):
        idxr, idxc, ga, gb, _, sema, semb, semc, _ = buf
        osub = pl.multiple_of(off // SUB, 8)
        pltpu.sync_copy(row_hbm.at[pl.ds(osub, NSUB1)], idxr)
        pltpu.sync_copy(col_hbm.at[pl.ds(osub, NSUB1)], idxc)
        for j in range(NSUB1):
            pltpu.async_copy(a_hbm.at[idxr.at[j]],
                             ga.at[pl.ds(j * SUB, SUB)], sema)
            pltpu.async_copy(b_hbm.at[idxc.at[j]],
                             gb.at[pl.ds(j * SUB, SUB)], semb)
            pltpu.async_copy(ones.at[pl.ds(0, SUB)], c_sh.at[idxc.at[j]],
                             semc, add=True)

    def drain_gathers(buf):
        _, _, ga, gb, _, sema, semb, semc, _ = buf
        pltpu.make_async_copy(a_hbm.at[pl.ds(0, C1)], ga, sema).wait()
        pltpu.make_async_copy(b_hbm.at[pl.ds(0, C1)], gb, semb).wait()
        pltpu.make_async_copy(c_hbm.at[0, pl.ds(0, C1)],
                              zflat.at[pl.ds(0, C1)], semc).wait()

    def compute(buf, off, first):
        _, _, ga, gb, htT, _, _, _, semw = buf

        @pl.when(jnp.logical_not(first))
        def _():
            pltpu.make_async_copy(htT, hp_hbm.at[:, pl.ds(0, C1)], semw).wait()

        @plsc.parallel_loop(0, C1, unroll=8)
        def _row(i):
            val = ga[i] + gb[i]
            plsc.store_scatter(htT, [iota16, jnp.full((F,), i, jnp.int32)],
                               val)

        pltpu.async_copy(htT, hp_hbm.at[:, pl.ds(off, C1)], semw)

    fire(bufs[0], base)

    @pl.loop(0, NCH1, step=2)
    def _chunk(ci):
        off = base + ci * C1
        fire(bufs[1], off + C1)
        drain_gathers(bufs[0])
        compute(bufs[0], off, ci == 0)

        @pl.when(ci + 2 < NCH1)
        def _():
            fire(bufs[0], off + 2 * C1)

        drain_gathers(bufs[1])
        compute(bufs[1], off + C1, ci == 0)

    for buf in bufs:
        htT, semw = buf[4], buf[8]
        pltpu.make_async_copy(htT, hp_hbm.at[:, pl.ds(0, C1)], semw).wait()
    plsc.subcore_barrier()
    pltpu.sync_copy(c_sh.at[pl.ds(cstart, 6256)],
                    c_hbm.at[scid, pl.ds(cstart, 6256)])


def _gather_hp(A, B, row3, col3):
    mesh = plsc.VectorSubcoreMesh(core_axis_name="c", subcore_axis_name="s")
    buf = [
        pltpu.VMEM((NSUB1, SUB), jnp.int32),
        pltpu.VMEM((NSUB1, SUB), jnp.int32),
        pltpu.VMEM((C1, F), jnp.float32),
        pltpu.VMEM((C1, F), jnp.float32),
        pltpu.VMEM((F, C1), jnp.float32),
    ]
    sems = [pltpu.SemaphoreType.DMA] * 4
    return pl.kernel(
        _k1_body,
        out_type=(jax.ShapeDtypeStruct((F, E), jnp.float32),
                  jax.ShapeDtypeStruct((NC, N), jnp.float32)),
        mesh=mesh,
        compiler_params=pltpu.CompilerParams(use_tc_tiling_on_sc=False, needs_layout_passes=False),
        scratch_types=buf + buf + [
            pltpu.VMEM((128,), jnp.float32),
            pltpu.VMEM((1024,), jnp.float32),
        ] + sems + sems + [
            pltpu.VMEM_SHARED((N,), jnp.float32),
        ],
    )(A, B, row3, col3)


EC = E // 128
CB = 200
NBC = EC // CB


def _k2a_body(hp_ref, ea_ref, wc_ref, h_ref, sum_ref, sq_ref):
    i = pl.program_id(0)

    @pl.when(i == 0)
    def _():
        sum_ref[...] = jnp.zeros_like(sum_ref)
        sq_ref[...] = jnp.zeros_like(sq_ref)

    wc = wc_ref[...]
    for c in range(CB):
        h = hp_ref[:, c, :] + jnp.dot(wc, ea_ref[:, c, :],
                                      preferred_element_type=jnp.float32)
        h_ref[:, c, :] = h
        sum_ref[...] += h
        sq_ref[...] += h * h


def _edge_h(hp3, ea3, wc):
    return pl.pallas_call(
        _k2a_body,
        grid=(NBC,),
        in_specs=[
            pl.BlockSpec((F, CB, 128), lambda i: (0, i, 0)),
            pl.BlockSpec((F, CB, 128), lambda i: (0, i, 0)),
            pl.BlockSpec((F, F), lambda i: (0, 0)),
        ],
        out_specs=[pl.BlockSpec((F, CB, 128), lambda i: (0, i, 0)),
                   pl.BlockSpec((F, 128), lambda i: (0, 0)),
                   pl.BlockSpec((F, 128), lambda i: (0, 0))],
        out_shape=[jax.ShapeDtypeStruct((F, EC, 128), jnp.float32),
                   jax.ShapeDtypeStruct((F, 128), jnp.float32),
                   jax.ShapeDtypeStruct((F, 128), jnp.float32)],
    )(hp3, ea3, wc)


def _k2b_body(h_ref, s_ref, t_ref, w2_ref, b2_ref, eo_ref):
    w2 = w2_ref[...]
    s = s_ref[...]
    t = t_ref[...]
    b2 = b2_ref[...]
    for c in range(CB):
        r = jnp.maximum(h_ref[:, c, :] * s + t, 0.0)
        eo_ref[:, c, :] = jnp.dot(w2, r,
                                  preferred_element_type=jnp.float32) + b2


def _edge_out(h3, scol, tcol, w2, b2col):
    return pl.pallas_call(
        _k2b_body,
        grid=(NBC,),
        in_specs=[
            pl.BlockSpec((F, CB, 128), lambda i: (0, i, 0)),
            pl.BlockSpec((F, 1), lambda i: (0, 0)),
            pl.BlockSpec((F, 1), lambda i: (0, 0)),
            pl.BlockSpec((F, F), lambda i: (0, 0)),
            pl.BlockSpec((F, 1), lambda i: (0, 0)),
        ],
        out_specs=pl.BlockSpec((F, CB, 128), lambda i: (0, i, 0)),
        out_shape=jax.ShapeDtypeStruct((F, EC, 128), jnp.float32),
    )(h3, scol, tcol, w2, b2col)


C3 = 1000
H3 = 500
NCH3 = EW // C3
NSUB3 = H3 // SUB


def _k3_body(h_hbm, col_hbm, s_hbm, t_hbm, sout_hbm,
             idxc, hv, hvT, sv_v, tv_v, sems, semh, s_sh):
    scid = lax.axis_index("c")
    sid = lax.axis_index("s")
    base = (scid * NS + sid) * EW
    iota16 = jax.lax.broadcasted_iota(jnp.int32, (F,), 0)

    pltpu.sync_copy(s_hbm, sv_v)
    pltpu.sync_copy(t_hbm, tv_v)
    sv = sv_v[...]
    tv = tv_v[...]

    @plsc.parallel_loop(0, H3, unroll=8)
    def _zfill(i):
        hv[i] = jnp.zeros((F,), jnp.float32)

    for z in range(13):
        zo = min(z * H3, NSL - H3)
        pltpu.sync_copy(hv, s_sh.at[pl.ds(sid * NSL + zo, H3)])
    plsc.subcore_barrier()

    pltpu.async_copy(h_hbm.at[:, pl.ds(base, C3)], hvT, semh)

    @pl.loop(0, NCH3)
    def _chunk(ci):
        off = base + ci * C3
        pltpu.make_async_copy(h_hbm.at[:, pl.ds(0, C3)], hvT, semh).wait()
        pltpu.sync_copy(col_hbm.at[pl.ds(pl.multiple_of(off // SUB, 8), C3 // SUB)],
                        idxc)
        for h2 in range(2):

            @plsc.parallel_loop(0, H3, unroll=8)
            def _row(i):
                hcol = plsc.load_gather(
                    hvT, [iota16, jnp.full((F,), h2 * H3 + i, jnp.int32)])
                hv[i] = jnp.maximum(hcol * sv + tv, 0.0)

            if h2 == 1:
                @pl.when(ci + 1 < NCH3)
                def _():
                    pltpu.async_copy(h_hbm.at[:, pl.ds(off + C3, C3)], hvT,
                                     semh)
            for j in range(NSUB3):
                pltpu.async_copy(hv.at[pl.ds(j * SUB, SUB)],
                                 s_sh.at[idxc.at[h2 * NSUB3 + j]], sems,
                                 add=True)
            pltpu.make_async_copy(sout_hbm.at[0, pl.ds(0, H3)], hv,
                                  sems).wait()

    plsc.subcore_barrier()
    pltpu.sync_copy(s_sh.at[pl.ds(sid * NSL, NSL)],
                    sout_hbm.at[scid, pl.ds(sid * NSL, NSL)])


def _scatter_r(h, col3, s16, t16):
    mesh = plsc.VectorSubcoreMesh(core_axis_name="c", subcore_axis_name="s")
    return pl.kernel(
        _k3_body,
        out_type=jax.ShapeDtypeStruct((NC, N, F), jnp.float32),
        mesh=mesh,
        compiler_params=pltpu.CompilerParams(use_tc_tiling_on_sc=False, needs_layout_passes=False),
        scratch_types=[
            pltpu.VMEM((C3 // SUB, SUB), jnp.int32),
            pltpu.VMEM((H3, F), jnp.float32),
            pltpu.VMEM((F, C3), jnp.float32),
            pltpu.VMEM((F,), jnp.float32),
            pltpu.VMEM((F,), jnp.float32),
            pltpu.SemaphoreType.DMA,
            pltpu.SemaphoreType.DMA,
            pltpu.VMEM_SHARED((N, F), jnp.float32),
        ],
    )(h, col3, s16, t16)


def _k4_body(x_ref, s0_ref, s1_ref, c0_ref, c1_ref, b2_ref, bt_ref, u_ref,
             mats_ref, vecs_ref,
             xo_ref, uo_ref,
             nsum, nsq, xa, sg, ecg, ncnt):
    i = pl.program_id(0)
    nwaT = mats_ref[0:16, :]
    nwbT = mats_ref[16:32, :]
    nw2T = mats_ref[32:48, :]
    ew2T = mats_ref[48:64, :]
    gwaT = mats_ref[64:80, :]
    gwbT = mats_ref[80:96, :]
    gwcT = mats_ref[96:112, :]
    n_b1 = vecs_ref[0:1, :]
    n_b2 = vecs_ref[1:2, :]
    n_g1 = vecs_ref[2:3, :]
    n_be1 = vecs_ref[3:4, :]
    e_b2 = vecs_ref[4:5, :]
    g_b1 = vecs_ref[5:6, :]
    g_g1 = vecs_ref[6:7, :]
    g_be1 = vecs_ref[7:8, :]

    bT = bt_ref[0]
    ohT = (lax.broadcasted_iota(jnp.int32, (G, BLKN), 0) == bT).astype(jnp.float32)

    def node_hidden():
        s_blk = s0_ref[...] + s1_ref[...]
        c_blk = jnp.broadcast_to(c0_ref[...] + c1_ref[...], (BLKN, F))
        e_aggr = ((jnp.dot(s_blk, ew2T, preferred_element_type=jnp.float32)
                   + c_blk * e_b2) / jnp.maximum(c_blk, 1.0))
        oh = (lax.broadcasted_iota(jnp.int32, (BLKN, G), 1)
              == b2_ref[...]).astype(jnp.float32)
        hn = (jnp.dot(x_ref[...], nwaT, preferred_element_type=jnp.float32)
              + jnp.dot(e_aggr, nwbT, preferred_element_type=jnp.float32)
              + jnp.dot(oh, jnp.dot(u_ref[...], mats_ref[112:128, :],
                                    preferred_element_type=jnp.float32),
                        preferred_element_type=jnp.float32)
              + n_b1)
        return hn, s_blk, c_blk

    @pl.when(i == 0)
    def _():
        nsum[...] = jnp.zeros_like(nsum)
        nsq[...] = jnp.zeros_like(nsq)
        xa[...] = jnp.zeros_like(xa)
        sg[...] = jnp.zeros_like(sg)
        ecg[...] = jnp.zeros_like(ecg)
        ncnt[...] = jnp.zeros_like(ncnt)

    @pl.when(i < NBN)
    def _phase_a():
        hn, s_blk, c_blk = node_hidden()
        nsum[...] += jnp.sum(hn, axis=0, keepdims=True)
        nsq[...] += jnp.sum(hn * hn, axis=0, keepdims=True)
        sg[...] += jnp.dot(ohT, s_blk, preferred_element_type=jnp.float32)
        ecg[...] += jnp.dot(ohT, c_blk, preferred_element_type=jnp.float32)
        ncnt[...] += jnp.dot(ohT, jnp.ones((BLKN, F), jnp.float32),
                             preferred_element_type=jnp.float32)

    @pl.when(i >= NBN)
    def _phase_b():
        hn, _, _ = node_hidden()
        mn = nsum[...] / N
        vr = nsq[...] / N - mn * mn
        sn = n_g1 * lax.rsqrt(vr + EPS)
        tn = n_be1 - mn * sn
        xo = jnp.dot(jnp.maximum(hn * sn + tn, 0.0), nw2T,
                     preferred_element_type=jnp.float32) + n_b2
        xo_ref[...] = xo
        w = jnp.where(i < 2 * NBN, 1.0, 0.0).astype(jnp.float32)
        xa[...] += w * jnp.dot(ohT, xo, preferred_element_type=jnp.float32)

    @pl.when(i == 2 * NBN)
    def _phase_g():
        x_aggr = xa[...] / jnp.maximum(ncnt[...], 1.0)
        e_aggr_g = ((jnp.dot(sg[...], ew2T, preferred_element_type=jnp.float32)
                     + ecg[...] * e_b2) / jnp.maximum(ecg[...], 1.0))
        go = (jnp.dot(u_ref[...], gwaT, preferred_element_type=jnp.float32)
              + jnp.dot(x_aggr, gwbT, preferred_element_type=jnp.float32)
              + jnp.dot(e_aggr_g, gwcT, preferred_element_type=jnp.float32)
              + g_b1)
        mg = jnp.sum(go, axis=0, keepdims=True) / G
        vg = jnp.sum(go * go, axis=0, keepdims=True) / G - mg * mg
        uo_ref[...] = jnp.maximum(
            (go - mg) * lax.rsqrt(vg + EPS) * g_g1 + g_be1, 0.0)


def _node_global(x, S, c2, batch2, batchT, u, mats, vecs):
    S0 = S[0]
    S1 = S[1]
    c0 = c2[0].reshape(N, 1)
    c1 = c2[1].reshape(N, 1)
    nb = lambda i: (i % NBN, 0)
    return pl.pallas_call(
        _k4_body,
        grid=(2 * NBN + 1,),
        in_specs=[
            pl.BlockSpec((BLKN, F), nb),
            pl.BlockSpec((BLKN, F), nb),
            pl.BlockSpec((BLKN, F), nb),
            pl.BlockSpec((BLKN, 1), nb),
            pl.BlockSpec((BLKN, 1), nb),
            pl.BlockSpec((BLKN, 1), nb),
            pl.BlockSpec((1, 1, BLKN), lambda i: (i % NBN, 0, 0)),
            pl.BlockSpec((G, F), lambda i: (0, 0)),
            pl.BlockSpec((128, F), lambda i: (0, 0)),
            pl.BlockSpec((8, F), lambda i: (0, 0)),
        ],
        out_specs=[pl.BlockSpec((BLKN, F), nb),
                   pl.BlockSpec((G, F), lambda i: (0, 0))],
        out_shape=[jax.ShapeDtypeStruct((N, F), jnp.float32),
                   jax.ShapeDtypeStruct((G, F), jnp.float32)],
        scratch_shapes=[
            pltpu.VMEM((1, F), jnp.float32),
            pltpu.VMEM((1, F), jnp.float32),
            pltpu.VMEM((G, F), jnp.float32),
            pltpu.VMEM((G, F), jnp.float32),
            pltpu.VMEM((G, F), jnp.float32),
            pltpu.VMEM((G, F), jnp.float32),
        ],
    )(x, S0, S1, c0, c1, batch2, batchT, u, mats, vecs)


def kernel(x, edge_index, edge_attr, u, batch,
           e_W1, e_b1, e_g1, e_be1, e_W2, e_b2,
           n_W1, n_b1, n_g1, n_be1, n_W2, n_b2,
           g_W1, g_b1, g_g1, g_be1):
    row = edge_index[0].astype(jnp.int32)
    col = edge_index[1].astype(jnp.int32)
    batch = batch.astype(jnp.int32)

    eWaT = e_W1[:, 0:16].T
    eWbT = e_W1[:, 16:32].T
    eWcT = e_W1[:, 32:48].T
    eWdT = e_W1[:, 48:64].T

    batch2 = batch.reshape(N, 1)
    batchT = batch.reshape(NBN, 1, BLKN)
    row3 = row.reshape(E // SUB, SUB)
    col3 = col.reshape(E // SUB, SUB)

    A, B = _node_tables(x, batch2, u, eWaT, eWbT, eWdT)

    hp, c2 = _gather_hp(A, B, row3, col3)

    ea3 = edge_attr.T.reshape(F, EC, 128)
    hp3 = hp.reshape(F, EC, 128)
    h3, hsum, hsq = _edge_h(hp3, ea3, e_W1[:, 32:48])

    hsum16 = jnp.sum(hsum, axis=1)
    hsq16 = jnp.sum(hsq, axis=1)
    mean = hsum16 / E
    var = hsq16 / E - mean * mean
    s16 = e_g1 * lax.rsqrt(var + EPS)
    t16 = e_be1 - mean * s16

    eo3 = _edge_out(h3, s16.reshape(F, 1), t16.reshape(F, 1), e_W2,
                    e_b2.reshape(F, 1))
    edge_out = eo3.reshape(F, E).T

    S = _scatter_r(h3.reshape(F, E), col3, s16, t16)

    mats = jnp.concatenate([
        n_W1[:, 0:16].T, n_W1[:, 16:32].T, n_W2.T, e_W2.T,
        g_W1[:, 0:16].T, g_W1[:, 16:32].T, g_W1[:, 32:48].T,
        n_W1[:, 32:48].T,
    ], axis=0)
    vecs = jnp.stack([n_b1, n_b2, n_g1, n_be1, e_b2, g_b1, g_g1, g_be1])
    x_out, u_out = _node_global(x, S, c2, batch2, batchT, u, mats, vecs)

    return (x_out, edge_out, u_out)
